# baseline (speedup 1.0000x reference)
import functools
import math

import jax
import jax.numpy as jnp
from jax import lax
from jax.experimental import pallas as pl
from jax.experimental.pallas import tpu as pltpu
from jax.experimental.pallas import tpu_sc as plsc

N = 10000
E = 160000
FEAT = 256
EMB = 512
HALF = 256
HID2 = 2 * EMB
L = 3
K = int(math.ceil(0.5 * N))

NP = 10240
KP = 5120
RBLK = 1000
NSUB = 16
NCORE = 2
NW = NSUB * NCORE
E_PER_W = 5008
E_PAD = NW * E_PER_W
NODE_CHUNK = 80
N_CHUNKS = N // NODE_CHUNK
G = 64
BCAP = 256
DROWS = KP // NSUB


def _score_body(x_ref, q_ref, o_ref):
    o_ref[...] = jnp.tanh(
        jnp.sum(x_ref[...] * q_ref[...], axis=1, keepdims=True))


def _score_call(x, q2d):
    return pl.pallas_call(
        _score_body,
        out_shape=jax.ShapeDtypeStruct((N, 1), jnp.float32),
    )(x, q2d)


def _select_body(s_ref, sm_ref, nm_ref):
    s = s_ref[...]
    bits = lax.bitcast_convert_type(s, jnp.int32)
    key = bits ^ ((bits >> 31) & jnp.int32(0x7FFFFFFF))
    kf = jnp.float32(K)

    def tbit(i, lo_u):
        b = 31 - i
        cand = lo_u | (jnp.int32(1) << b)
        t_s = cand ^ jnp.int32(-2147483648)
        cnt = jnp.sum((key >= t_s).astype(jnp.float32))
        return jnp.where(cnt >= kf, cand, lo_u)

    lo_u = lax.fori_loop(0, 32, tbit, jnp.int32(0))
    t_star = lo_u ^ jnp.int32(-2147483648)
    cnt_gt = jnp.sum((key > t_star).astype(jnp.float32))
    r = kf - cnt_gt

    rr = lax.broadcasted_iota(jnp.int32, s.shape, 0)
    cc = lax.broadcasted_iota(jnp.int32, s.shape, 1)
    idx = rr * 128 + cc
    eq = key == t_star

    def mbit(i, lo_m):
        b = 14 - i
        cand = lo_m | (jnp.int32(1) << b)
        ecnt = jnp.sum((eq & (idx < cand)).astype(jnp.float32))
        return jnp.where(ecnt <= r, cand, lo_m)

    m_star = lax.fori_loop(0, 15, mbit, jnp.int32(0))
    mask = (key > t_star) | (eq & (idx < m_star))
    maskf = mask.astype(jnp.float32)
    sm_ref[...] = maskf * s

    tri = (lax.broadcasted_iota(jnp.int32, (128, 128), 0)
           < lax.broadcasted_iota(jnp.int32, (128, 128), 1)).astype(jnp.float32)
    pre_in_row = jnp.dot(maskf, tri, preferred_element_type=jnp.float32)
    rows = s.shape[0]
    rs = jnp.sum(maskf, axis=1, keepdims=True)
    plow = (lax.broadcasted_iota(jnp.int32, (rows, rows), 1)
            < lax.broadcasted_iota(jnp.int32, (rows, rows), 0)
            ).astype(jnp.float32)
    row_off = jnp.dot(plow, rs, preferred_element_type=jnp.float32)
    c2d = row_off + pre_in_row
    nm_ref[...] = jnp.where(mask, c2d.astype(jnp.int32), jnp.int32(-1))


def _select_call(scorep):
    return pl.pallas_call(
        _select_body,
        out_shape=[
            jax.ShapeDtypeStruct((NP // 128, 128), jnp.float32),
            jax.ShapeDtypeStruct((NP // 128, 128), jnp.int32),
        ],
    )(scorep)


def _prescale_body(x_ref, sm_ref, o_ref):
    o_ref[...] = x_ref[...] * sm_ref[...]


def _prescale_call(x, sm_col):
    return pl.pallas_call(
        _prescale_body,
        grid=(5,),
        in_specs=[
            pl.BlockSpec((2000, FEAT), lambda i: (i, 0)),
            pl.BlockSpec((2000, 1), lambda i: (i, 0)),
        ],
        out_specs=pl.BlockSpec((2000, FEAT), lambda i: (i, 0)),
        out_shape=jax.ShapeDtypeStruct((N, FEAT), jnp.float32),
    )(x, sm_col)


def _compact_body(xsm_hbm, nm_hbm, rowp_hbm, colp_hbm,
                  xs_out, elist_out, ecnt_out,
                  nm_v, rowb, colb, bbuf, cntb, xrows, slotb, sem):
    cid = lax.axis_index("c")
    sid = lax.axis_index("s")
    w = sid * NCORE + cid
    pltpu.sync_copy(nm_hbm, nm_v)

    def do_chunk(ch):
        base = ch * NODE_CHUNK
        pltpu.sync_copy(xsm_hbm.at[pl.ds(base, NODE_CHUNK)], xrows)

        def lane(v, c2):
            nm16 = nm_v[pl.ds(base + v * 16, 16)]
            slotb[pl.ds(v * 16, 16)] = jnp.where(
                nm16 >= 0, nm16, jnp.int32(K) + w)
            return c2

        lax.fori_loop(0, NODE_CHUNK // 16, lane, 0)
        pltpu.sync_copy(xrows, xs_out.at[slotb])

    def chunk_loop(k, c2):
        ch = w + NW * k

        @pl.when(ch < N_CHUNKS)
        def _():
            do_chunk(ch)
        return c2

    lax.fori_loop(0, (N_CHUNKS + NW - 1) // NW, chunk_loop, 0)

    ebase = w * E_PER_W
    pltpu.sync_copy(rowp_hbm.at[pl.ds(ebase, E_PER_W)], rowb)
    pltpu.sync_copy(colp_hbm.at[pl.ds(ebase, E_PER_W)], colb)

    lane16 = lax.iota(jnp.int32, 16)

    def egroup(g, cntv):
        r16 = rowb[pl.ds(g * 16, 16)]
        c16 = colb[pl.ds(g * 16, 16)]
        mr = plsc.load_gather(nm_v, [r16])
        mc = plsc.load_gather(nm_v, [c16])
        keep = (mr >= 0) & (mc >= 0)
        packed = mr | (mc << 13)
        bucket = lax.div(mc, jnp.int32(DROWS))
        for b in range(NSUB):
            keepb = keep & (bucket == b)
            plsc.store_compressed(
                bbuf.at[b, pl.ds(cntv[b], 16)], packed, mask=keepb)
            onehot = (lane16 == b).astype(jnp.int32)
            cntv = cntv + onehot * jnp.sum(keepb.astype(jnp.int32))
        return cntv

    cntv = lax.fori_loop(0, E_PER_W // 16, egroup,
                         jnp.zeros((16,), jnp.int32))
    cntb[pl.ds(0, 16)] = cntv
    pltpu.sync_copy(bbuf, elist_out.at[w])
    pltpu.sync_copy(cntb, ecnt_out.at[pl.ds(w * 16, 16)])


def _make_compact_call():
    mesh = plsc.VectorSubcoreMesh(core_axis_name="c", subcore_axis_name="s")
    return pl.kernel(
        _compact_body,
        out_type=[
            jax.ShapeDtypeStruct((KP, FEAT), jnp.float32),
            jax.ShapeDtypeStruct((NW, NSUB, BCAP), jnp.int32),
            jax.ShapeDtypeStruct((NW * 16,), jnp.int32),
        ],
        mesh=mesh,
        scratch_types=[
            pltpu.VMEM((NP,), jnp.int32),
            pltpu.VMEM((E_PER_W,), jnp.int32),
            pltpu.VMEM((E_PER_W,), jnp.int32),
            pltpu.VMEM((NSUB, BCAP), jnp.int32),
            pltpu.VMEM((16,), jnp.int32),
            pltpu.VMEM((NODE_CHUNK, FEAT), jnp.float32),
            pltpu.VMEM((NODE_CHUNK,), jnp.int32),
            pltpu.SemaphoreType.DMA,
        ],
        compiler_params=pltpu.CompilerParams(needs_layout_passes=False),
    )


def _fc_body(x_ref, w_ref, hA, hB, rA, rB):
    z = jnp.dot(x_ref[...], w_ref[...], preferred_element_type=jnp.float32)
    zr = jnp.maximum(z, 0.0)
    hA[...] = z[:, :HALF]
    hB[...] = z[:, HALF:]
    rA[...] = zr[:, :HALF]
    rB[...] = zr[:, HALF:]


def _fc_call(xs, w_fc):
    half_spec = pl.BlockSpec((RBLK, HALF), lambda i: (i, 0))
    return pl.pallas_call(
        _fc_body,
        grid=(K // RBLK,),
        in_specs=[
            pl.BlockSpec((RBLK, FEAT), lambda i: (i, 0)),
            pl.BlockSpec((FEAT, EMB), lambda i: (0, 0)),
        ],
        out_specs=[half_spec] * 4,
        out_shape=[jax.ShapeDtypeStruct((K, HALF), jnp.float32)] * 4,
    )(xs, w_fc)


def _agg_body(elist_hbm, ecnt_hbm, zeros_hbm, rhA, rhB, oA, oB,
              lbuf, cball, srcb, offb, rows_v, acc, sem):
    cid = lax.axis_index("c")
    sid = lax.axis_index("s")
    zero16i = jnp.zeros((16,), jnp.int32)

    pltpu.sync_copy(ecnt_hbm, cball.at[pl.ds(0, NW * 16)])
    pltpu.sync_copy(zeros_hbm, acc)

    def run(rh_t, out_t):
        def wloop(w, c0):
            pltpu.sync_copy(elist_hbm.at[w, sid], lbuf)
            cnt = cball[pl.ds(w * 16 + sid, 16)][0]
            for t in range(G // 16):
                lbuf[pl.ds(cnt + 16 * t, 16)] = zero16i
            ng = lax.div(cnt + (G - 1), jnp.int32(G))

            def grp(g, c2):
                for sub in range(G // 16):
                    p16 = lbuf[pl.ds(g * G + sub * 16, 16)]
                    srcb[pl.ds(sub * 16, 16)] = p16 & jnp.int32(0x1FFF)
                    offb[pl.ds(sub * 16, 16)] = (
                        (p16 >> 13) - sid * DROWS) * jnp.int32(HALF)
                pltpu.async_copy(rh_t.at[srcb], rows_v, sem).wait()
                for sub in range(G // 16):
                    off16 = offb[pl.ds(sub * 16, 16)]
                    for e in range(16):
                        j = sub * 16 + e

                        @pl.when(g * G + j < cnt)
                        def _(j=j, e=e, off16=off16):
                            base = off16[e]
                            for cc in range(HALF // 16):
                                plsc.addupdate(
                                    acc.at[pl.ds(base + cc * 16, 16)],
                                    rows_v[j, pl.ds(cc * 16, 16)])
                return c2

            lax.fori_loop(0, ng, grp, 0)
            return c0

        lax.fori_loop(0, NW, wloop, 0)
        pltpu.sync_copy(acc, out_t.at[pl.ds(sid * DROWS * HALF, DROWS * HALF)])

    @pl.when(cid == 0)
    def _():
        run(rhA, oA)

    @pl.when(cid == 1)
    def _():
        run(rhB, oB)


def _make_agg_call():
    mesh = plsc.VectorSubcoreMesh(core_axis_name="c", subcore_axis_name="s")
    return pl.kernel(
        _agg_body,
        out_type=[jax.ShapeDtypeStruct((KP * HALF,), jnp.float32)] * 2,
        mesh=mesh,
        scratch_types=[
            pltpu.VMEM((BCAP,), jnp.int32),
            pltpu.VMEM((NW * 16 + 16,), jnp.int32),
            pltpu.VMEM((G,), jnp.int32),
            pltpu.VMEM((G,), jnp.int32),
            pltpu.VMEM((G, HALF), jnp.float32),
            pltpu.VMEM((DROWS * HALF,), jnp.float32),
            pltpu.SemaphoreType.DMA,
        ],
        compiler_params=pltpu.CompilerParams(needs_layout_passes=False),
    )


def _layer_a_body(hA, hB, aA, aB, w_ref, b_ref, eps_ref, z1_ref, st_ref):
    i = pl.program_id(0)
    h = jnp.concatenate([hA[...], hB[...]], axis=1)
    a = jnp.concatenate([aA[...], aB[...]], axis=1)
    zin = h * (1.0 + eps_ref[0]) + a
    z1 = jnp.dot(zin, w_ref[...], preferred_element_type=jnp.float32)
    z1 = z1 + b_ref[...]
    z1_ref[...] = z1

    @pl.when(i == 0)
    def _():
        st_ref[...] = jnp.zeros_like(st_ref)

    st_ref[0:1, :] = st_ref[0:1, :] + jnp.sum(z1, axis=0, keepdims=True)
    st_ref[1:2, :] = st_ref[1:2, :] + jnp.sum(z1 * z1, axis=0, keepdims=True)


def _layer_a_call(hA, hB, aA, aB, w1l, b1l, epsl):
    half_spec = pl.BlockSpec((RBLK, HALF), lambda i: (i, 0))
    return pl.pallas_call(
        _layer_a_body,
        grid=(K // RBLK,),
        in_specs=[half_spec] * 4 + [
            pl.BlockSpec((EMB, HID2), lambda i: (0, 0)),
            pl.BlockSpec((1, HID2), lambda i: (0, 0)),
            pl.BlockSpec(memory_space=pltpu.SMEM),
        ],
        out_specs=[
            pl.BlockSpec((RBLK, HID2), lambda i: (i, 0)),
            pl.BlockSpec((8, HID2), lambda i: (0, 0)),
        ],
        out_shape=[
            jax.ShapeDtypeStruct((K, HID2), jnp.float32),
            jax.ShapeDtypeStruct((8, HID2), jnp.float32),
        ],
    )(hA, hB, aA, aB, w1l, b1l, epsl)


def _layer_b_body(z1_ref, st_ref, g_ref, be_ref, w_ref, b_ref,
                  z2_ref, st2_ref):
    i = pl.program_id(0)
    kf = jnp.float32(K)
    mean = st_ref[0:1, :] / kf
    var = st_ref[1:2, :] / kf - mean * mean
    z1 = z1_ref[...]
    xb = g_ref[...] * (z1 - mean) / jnp.sqrt(var + 1e-5) + be_ref[...]
    y = jnp.maximum(xb, 0.0)
    z2 = jnp.dot(y, w_ref[...], preferred_element_type=jnp.float32)
    z2 = z2 + b_ref[...]
    z2_ref[...] = z2

    @pl.when(i == 0)
    def _():
        st2_ref[...] = jnp.zeros_like(st2_ref)

    st2_ref[0:1, :] = st2_ref[0:1, :] + jnp.sum(z2, axis=0, keepdims=True)
    st2_ref[1:2, :] = st2_ref[1:2, :] + jnp.sum(z2 * z2, axis=0, keepdims=True)


def _layer_b_call(z1, st, g1l, be1l, w2l, b2l):
    return pl.pallas_call(
        _layer_b_body,
        grid=(K // RBLK,),
        in_specs=[
            pl.BlockSpec((RBLK, HID2), lambda i: (i, 0)),
            pl.BlockSpec((8, HID2), lambda i: (0, 0)),
            pl.BlockSpec((1, HID2), lambda i: (0, 0)),
            pl.BlockSpec((1, HID2), lambda i: (0, 0)),
            pl.BlockSpec((HID2, EMB), lambda i: (0, 0)),
            pl.BlockSpec((1, EMB), lambda i: (0, 0)),
        ],
        out_specs=[
            pl.BlockSpec((RBLK, EMB), lambda i: (i, 0)),
            pl.BlockSpec((8, EMB), lambda i: (0, 0)),
        ],
        out_shape=[
            jax.ShapeDtypeStruct((K, EMB), jnp.float32),
            jax.ShapeDtypeStruct((8, EMB), jnp.float32),
        ],
    )(z1, st, g1l, be1l, w2l, b2l)


def _layer_c_body(z2_ref, st_ref, g_ref, be_ref, hA, hB):
    kf = jnp.float32(K)
    mean = st_ref[0:1, :] / kf
    var = st_ref[1:2, :] / kf - mean * mean
    xb = g_ref[...] * (z2_ref[...] - mean) / jnp.sqrt(var + 1e-5) + be_ref[...]
    h = jnp.maximum(xb, 0.0)
    hA[...] = h[:, :HALF]
    hB[...] = h[:, HALF:]


def _layer_c_call(z2, st2, gbnl, bbnl):
    half_spec = pl.BlockSpec((RBLK, HALF), lambda i: (i, 0))
    return pl.pallas_call(
        _layer_c_body,
        grid=(K // RBLK,),
        in_specs=[
            pl.BlockSpec((RBLK, EMB), lambda i: (i, 0)),
            pl.BlockSpec((8, EMB), lambda i: (0, 0)),
            pl.BlockSpec((1, EMB), lambda i: (0, 0)),
            pl.BlockSpec((1, EMB), lambda i: (0, 0)),
        ],
        out_specs=[half_spec] * 2,
        out_shape=[jax.ShapeDtypeStruct((K, HALF), jnp.float32)] * 2,
    )(z2, st2, gbnl, bbnl)


def _layer_pool_body(z2_ref, st_ref, g_ref, be_ref, pool_ref):
    i = pl.program_id(0)
    kf = jnp.float32(K)
    mean = st_ref[0:1, :] / kf
    var = st_ref[1:2, :] / kf - mean * mean
    h = g_ref[...] * (z2_ref[...] - mean) / jnp.sqrt(var + 1e-5) + be_ref[...]

    @pl.when(i == 0)
    def _():
        pool_ref[...] = jnp.zeros_like(pool_ref)

    pool_ref[0:1, :] = pool_ref[0:1, :] + jnp.sum(h, axis=0, keepdims=True)


def _layer_pool_call(z2, st2, gbnl, bbnl):
    return pl.pallas_call(
        _layer_pool_body,
        grid=(K // RBLK,),
        in_specs=[
            pl.BlockSpec((RBLK, EMB), lambda i: (i, 0)),
            pl.BlockSpec((8, EMB), lambda i: (0, 0)),
            pl.BlockSpec((1, EMB), lambda i: (0, 0)),
            pl.BlockSpec((1, EMB), lambda i: (0, 0)),
        ],
        out_specs=pl.BlockSpec((8, EMB), lambda i: (0, 0)),
        out_shape=jax.ShapeDtypeStruct((8, EMB), jnp.float32),
    )(z2, st2, gbnl, bbnl)


def _head_body(pool_ref, text_ref, wp1_ref, bp1_ref, wp2_ref, bp2_ref, o_ref):
    p = pool_ref[0:1, :] * (1.0 / jnp.float32(K))
    a1 = wp1_ref[0:EMB, :]
    a2 = wp1_ref[EMB:EMB + FEAT, :]
    r = jnp.dot(p, a1, preferred_element_type=jnp.float32)
    r = r + jnp.dot(text_ref[...], a2, preferred_element_type=jnp.float32)
    r = jnp.maximum(r + bp1_ref[...], 0.0)
    o = jnp.dot(r, wp2_ref[...], preferred_element_type=jnp.float32)
    o_ref[...] = o + bp2_ref[...]


def _head_call(pool, text_emb, wp1, bp1, wp2, bp2):
    return pl.pallas_call(
        _head_body,
        out_shape=jax.ShapeDtypeStruct((1, 2), jnp.float32),
    )(pool, text_emb, wp1, bp1, wp2, bp2)


def kernel(text_emb, demand_kg_emb, x, edge_index, W_fc, eps, W1, b1, g1,
           be1, W2, b2, gbn, bbn, Wp1, bp1, Wp2, bp2):
    score = _score_call(x, demand_kg_emb)
    scorep = jnp.concatenate(
        [score.reshape(N), jnp.full((NP - N,), -2.0, jnp.float32)]
    ).reshape(NP // 128, 128)
    sm2d, nm2d = _select_call(scorep)
    sm_col = sm2d.reshape(NP)[:N].reshape(N, 1)
    nm_flat = nm2d.reshape(NP)

    xsm = _prescale_call(x, sm_col)

    row = edge_index[0].astype(jnp.int32)
    col = edge_index[1].astype(jnp.int32)
    pad = jnp.full((E_PAD - E,), jnp.int32(N))
    rowp = jnp.concatenate([row, pad])
    colp = jnp.concatenate([col, pad])

    compact_fn = _make_compact_call()
    xs, elist, ecnt = compact_fn(xsm, nm_flat, rowp, colp)

    hA, hB, rA, rB = _fc_call(xs, W_fc)

    zeros_sc = jnp.zeros((DROWS * HALF,), jnp.float32)
    agg_fn = _make_agg_call()

    for l in range(L):
        aAf, aBf = agg_fn(elist, ecnt, zeros_sc, rA, rB)
        aA = aAf.reshape(KP, HALF)
        aB = aBf.reshape(KP, HALF)
        z1, st1 = _layer_a_call(hA, hB, aA, aB, W1[l],
                                b1[l].reshape(1, HID2), eps[l].reshape(1))
        z2, st2 = _layer_b_call(z1, st1, g1[l].reshape(1, HID2),
                                be1[l].reshape(1, HID2), W2[l],
                                b2[l].reshape(1, EMB))
        if l < L - 1:
            hA, hB = _layer_c_call(z2, st2, gbn[l].reshape(1, EMB),
                                   bbn[l].reshape(1, EMB))
            rA, rB = hA, hB
        else:
            pool = _layer_pool_call(z2, st2, gbn[l].reshape(1, EMB),
                                    bbn[l].reshape(1, EMB))

    return _head_call(pool, text_emb, Wp1, bp1.reshape(1, -1), Wp2,
                      bp2.reshape(1, 2))


# speedup vs baseline: 3.4737x; 3.4737x over previous
import functools
import math

import jax
import jax.numpy as jnp
from jax import lax
from jax.experimental import pallas as pl
from jax.experimental.pallas import tpu as pltpu
from jax.experimental.pallas import tpu_sc as plsc

N = 10000
E = 160000
FEAT = 256
EMB = 512
HALF = 256
HID2 = 2 * EMB
L = 3
K = int(math.ceil(0.5 * N))

NP = 10240
KP = 5120
RBLK = 1000
NSUB = 16
NCORE = 2
NW = NSUB * NCORE
E_PER_W = 5008
E_PAD = NW * E_PER_W
NODE_CHUNK = 80
N_CHUNKS = N // NODE_CHUNK
G = 64
BCAP = 256
DROWS = KP // NSUB
MCAP = 2048


def _score_body(x_ref, q_ref, o_ref):
    o_ref[...] = jnp.tanh(
        jnp.sum(x_ref[...] * q_ref[...], axis=1, keepdims=True))


def _score_call(x, q2d):
    return pl.pallas_call(
        _score_body,
        out_shape=jax.ShapeDtypeStruct((N, 1), jnp.float32),
    )(x, q2d)


def _select_body(s_ref, sm_ref, nm_ref):
    s = s_ref[...]
    bits = lax.bitcast_convert_type(s, jnp.int32)
    key = bits ^ ((bits >> 31) & jnp.int32(0x7FFFFFFF))
    kf = jnp.float32(K)

    def tbit(i, lo_u):
        b = 31 - i
        cand = lo_u | (jnp.int32(1) << b)
        t_s = cand ^ jnp.int32(-2147483648)
        cnt = jnp.sum((key >= t_s).astype(jnp.float32))
        return jnp.where(cnt >= kf, cand, lo_u)

    lo_u = lax.fori_loop(0, 32, tbit, jnp.int32(0))
    t_star = lo_u ^ jnp.int32(-2147483648)
    cnt_gt = jnp.sum((key > t_star).astype(jnp.float32))
    r = kf - cnt_gt

    rr = lax.broadcasted_iota(jnp.int32, s.shape, 0)
    cc = lax.broadcasted_iota(jnp.int32, s.shape, 1)
    idx = rr * 128 + cc
    eq = key == t_star

    def mbit(i, lo_m):
        b = 14 - i
        cand = lo_m | (jnp.int32(1) << b)
        ecnt = jnp.sum((eq & (idx < cand)).astype(jnp.float32))
        return jnp.where(ecnt <= r, cand, lo_m)

    m_star = lax.fori_loop(0, 15, mbit, jnp.int32(0))
    mask = (key > t_star) | (eq & (idx < m_star))
    maskf = mask.astype(jnp.float32)
    sm_ref[...] = maskf * s

    tri = (lax.broadcasted_iota(jnp.int32, (128, 128), 0)
           < lax.broadcasted_iota(jnp.int32, (128, 128), 1)).astype(jnp.float32)
    pre_in_row = jnp.dot(maskf, tri, preferred_element_type=jnp.float32)
    rows = s.shape[0]
    rs = jnp.sum(maskf, axis=1, keepdims=True)
    plow = (lax.broadcasted_iota(jnp.int32, (rows, rows), 1)
            < lax.broadcasted_iota(jnp.int32, (rows, rows), 0)
            ).astype(jnp.float32)
    row_off = jnp.dot(plow, rs, preferred_element_type=jnp.float32)
    c2d = row_off + pre_in_row
    nm_ref[...] = jnp.where(mask, c2d.astype(jnp.int32), jnp.int32(-1))


def _select_call(scorep):
    return pl.pallas_call(
        _select_body,
        out_shape=[
            jax.ShapeDtypeStruct((NP // 128, 128), jnp.float32),
            jax.ShapeDtypeStruct((NP // 128, 128), jnp.int32),
        ],
    )(scorep)


def _prescale_body(x_ref, sm_ref, o_ref):
    o_ref[...] = x_ref[...] * sm_ref[...]


def _prescale_call(x, sm_col):
    return pl.pallas_call(
        _prescale_body,
        grid=(5,),
        in_specs=[
            pl.BlockSpec((2000, FEAT), lambda i: (i, 0)),
            pl.BlockSpec((2000, 1), lambda i: (i, 0)),
        ],
        out_specs=pl.BlockSpec((2000, FEAT), lambda i: (i, 0)),
        out_shape=jax.ShapeDtypeStruct((N, FEAT), jnp.float32),
    )(x, sm_col)


def _compact_body(xsm_hbm, nm_hbm, rowp_hbm, colp_hbm,
                  xs_out, elist_out, ecnt_out,
                  nm_v, rowb, colb, bbuf, cntb, xrows, slotb, sem):
    cid = lax.axis_index("c")
    sid = lax.axis_index("s")
    w = sid * NCORE + cid
    pltpu.sync_copy(nm_hbm, nm_v)

    def do_chunk(ch):
        base = ch * NODE_CHUNK
        pltpu.sync_copy(xsm_hbm.at[pl.ds(base, NODE_CHUNK)], xrows)

        def lane(v, c2):
            nm16 = nm_v[pl.ds(base + v * 16, 16)]
            slotb[pl.ds(v * 16, 16)] = jnp.where(
                nm16 >= 0, nm16, jnp.int32(K) + w)
            return c2

        lax.fori_loop(0, NODE_CHUNK // 16, lane, 0)
        pltpu.sync_copy(xrows, xs_out.at[slotb])

    def chunk_loop(k, c2):
        ch = w + NW * k

        @pl.when(ch < N_CHUNKS)
        def _():
            do_chunk(ch)
        return c2

    lax.fori_loop(0, (N_CHUNKS + NW - 1) // NW, chunk_loop, 0)

    ebase = w * E_PER_W
    pltpu.sync_copy(rowp_hbm.at[pl.ds(ebase, E_PER_W)], rowb)
    pltpu.sync_copy(colp_hbm.at[pl.ds(ebase, E_PER_W)], colb)

    lane16 = lax.iota(jnp.int32, 16)

    def egroup(g, cntv):
        r16 = rowb[pl.ds(g * 16, 16)]
        c16 = colb[pl.ds(g * 16, 16)]
        mr = plsc.load_gather(nm_v, [r16])
        mc = plsc.load_gather(nm_v, [c16])
        keep = (mr >= 0) & (mc >= 0)
        packed = mr | (mc << 13)
        bucket = lax.div(mc, jnp.int32(DROWS))
        for b in range(NSUB):
            keepb = keep & (bucket == b)
            plsc.store_compressed(
                bbuf.at[b, pl.ds(cntv[b], 16)], packed, mask=keepb)
            onehot = (lane16 == b).astype(jnp.int32)
            cntv = cntv + onehot * jnp.sum(keepb.astype(jnp.int32))
        return cntv

    cntv = lax.fori_loop(0, E_PER_W // 16, egroup,
                         jnp.zeros((16,), jnp.int32))
    cntb[pl.ds(0, 16)] = cntv
    pltpu.sync_copy(bbuf, elist_out.at[w])
    pltpu.sync_copy(cntb, ecnt_out.at[pl.ds(w * 16, 16)])


def _make_compact_call():
    mesh = plsc.VectorSubcoreMesh(core_axis_name="c", subcore_axis_name="s")
    return pl.kernel(
        _compact_body,
        out_type=[
            jax.ShapeDtypeStruct((KP, FEAT), jnp.float32),
            jax.ShapeDtypeStruct((NW, NSUB, BCAP), jnp.int32),
            jax.ShapeDtypeStruct((NW * 16,), jnp.int32),
        ],
        mesh=mesh,
        scratch_types=[
            pltpu.VMEM((NP,), jnp.int32),
            pltpu.VMEM((E_PER_W,), jnp.int32),
            pltpu.VMEM((E_PER_W,), jnp.int32),
            pltpu.VMEM((NSUB, BCAP), jnp.int32),
            pltpu.VMEM((16,), jnp.int32),
            pltpu.VMEM((NODE_CHUNK, FEAT), jnp.float32),
            pltpu.VMEM((NODE_CHUNK,), jnp.int32),
            pltpu.SemaphoreType.DMA,
        ],
        compiler_params=pltpu.CompilerParams(needs_layout_passes=False),
    )


def _fc_body(x_ref, w_ref, hA, hB, rA, rB):
    z = jnp.dot(x_ref[...], w_ref[...], preferred_element_type=jnp.float32)
    zr = jnp.maximum(z, 0.0)
    hA[...] = z[:, :HALF]
    hB[...] = z[:, HALF:]
    rA[...] = zr[:, :HALF]
    rB[...] = zr[:, HALF:]


def _fc_call(xs, w_fc):
    half_spec = pl.BlockSpec((RBLK, HALF), lambda i: (i, 0))
    return pl.pallas_call(
        _fc_body,
        grid=(K // RBLK,),
        in_specs=[
            pl.BlockSpec((RBLK, FEAT), lambda i: (i, 0)),
            pl.BlockSpec((FEAT, EMB), lambda i: (0, 0)),
        ],
        out_specs=[half_spec] * 4,
        out_shape=[jax.ShapeDtypeStruct((K, HALF), jnp.float32)] * 4,
    )(xs, w_fc)


def _merge_body(elist_hbm, ecnt_hbm, mlist_out, mcnt_out,
                lbuf, cball, big, cntb, sem):
    cid = lax.axis_index("c")
    sid = lax.axis_index("s")
    pltpu.sync_copy(ecnt_hbm, cball.at[pl.ds(0, NW * 16)])
    lane16 = lax.iota(jnp.int32, 16)

    def wloop(wi, off):
        w = cid * NSUB + wi
        pltpu.sync_copy(elist_hbm.at[w, sid], lbuf)
        cnt = cball[pl.ds(w * 16 + sid, 16)][0]
        ng = lax.div(cnt + 15, jnp.int32(16))

        def cp(t, c2):
            big[pl.ds(off + t * 16, 16)] = lbuf[pl.ds(t * 16, 16)]
            return c2

        lax.fori_loop(0, ng, cp, 0)
        return off + cnt

    off = lax.fori_loop(0, NSUB, wloop, jnp.int32(0))
    padv = (lane16 * 0 + (sid * DROWS + DROWS)) << 13
    for t in range(G // 16):
        big[pl.ds(off + t * 16, 16)] = padv
    cntb[pl.ds(0, 16)] = lane16 * 0 + off
    seg = sid * 2 + cid
    pltpu.sync_copy(big, mlist_out.at[seg])
    pltpu.sync_copy(cntb, mcnt_out.at[pl.ds(seg * 16, 16)])


def _make_merge_call():
    mesh = plsc.VectorSubcoreMesh(core_axis_name="c", subcore_axis_name="s")
    return pl.kernel(
        _merge_body,
        out_type=[
            jax.ShapeDtypeStruct((NW, MCAP), jnp.int32),
            jax.ShapeDtypeStruct((NW * 16,), jnp.int32),
        ],
        mesh=mesh,
        scratch_types=[
            pltpu.VMEM((BCAP,), jnp.int32),
            pltpu.VMEM((NW * 16 + 16,), jnp.int32),
            pltpu.VMEM((MCAP,), jnp.int32),
            pltpu.VMEM((16,), jnp.int32),
            pltpu.SemaphoreType.DMA,
        ],
        compiler_params=pltpu.CompilerParams(needs_layout_passes=False),
    )


def _agg_body(mlist_hbm, mcnt_hbm, zeros_hbm, rhA, rhB, oA, oB,
              lbuf2, cball, srcb, offb, rows0, rows1, acc, sem0, sem1):
    cid = lax.axis_index("c")
    sid = lax.axis_index("s")

    pltpu.sync_copy(mcnt_hbm, cball.at[pl.ds(0, NW * 16)])
    pltpu.sync_copy(zeros_hbm, acc.at[pl.ds(0, DROWS * HALF)])
    pltpu.sync_copy(mlist_hbm.at[sid * 2], lbuf2.at[pl.ds(0, MCAP)])
    pltpu.sync_copy(mlist_hbm.at[sid * 2 + 1], lbuf2.at[pl.ds(MCAP, MCAP)])

    def unpack(segbase, g, p):
        def sub_loop(sub, c):
            p16 = lbuf2[pl.ds(segbase + g * G + sub * 16, 16)]
            srcb[pl.ds(p * G + sub * 16, 16)] = p16 & jnp.int32(0x1FFF)
            offb[pl.ds(p * G + sub * 16, 16)] = (
                (p16 >> 13) - sid * DROWS) * jnp.int32(HALF)
            return c

        lax.fori_loop(0, G // 16, sub_loop, 0)

    def accumulate(p, rows_ref):
        def sub_loop(sub, c):
            off16 = offb[pl.ds(p * G + sub * 16, 16)]
            for e in range(16):
                base = off16[e]
                for chunk in range(HALF // 16):
                    plsc.addupdate(
                        acc.at[pl.ds(base + chunk * 16, 16)],
                        rows_ref[sub * 16 + e, pl.ds(chunk * 16, 16)])
            return c

        lax.fori_loop(0, G // 16, sub_loop, 0)

    def run(rh_t, out_t):
        def do_seg(seg, segbase):
            cnt = cball[pl.ds(sid * 32 + seg * 16, 16)][0]
            ng = lax.div(cnt + (G - 1), jnp.int32(G))

            @pl.when(ng > 0)
            def _():
                unpack(segbase, 0, 0)
                pltpu.async_copy(rh_t.at[srcb.at[pl.ds(0, G)]], rows0, sem0)

                def step(g, rcur, scur, rnext, snext, pcur, pnext):
                    @pl.when(g + 1 < ng)
                    def _():
                        unpack(segbase, g + 1, pnext)
                        pltpu.async_copy(
                            rh_t.at[srcb.at[pl.ds(pnext * G, G)]],
                            rnext, snext)

                    pltpu.make_async_copy(
                        rh_t.at[srcb.at[pl.ds(pcur * G, G)]],
                        rcur, scur).wait()
                    accumulate(pcur, rcur)

                def body(g, c2):
                    @pl.when((g & 1) == 0)
                    def _():
                        step(g, rows0, sem0, rows1, sem1, 0, 1)

                    @pl.when((g & 1) == 1)
                    def _():
                        step(g, rows1, sem1, rows0, sem0, 1, 0)

                    return c2

                lax.fori_loop(0, ng, body, 0)

        do_seg(0, 0)
        do_seg(1, MCAP)
        pltpu.sync_copy(acc.at[pl.ds(0, DROWS * HALF)],
                        out_t.at[pl.ds(sid * DROWS * HALF, DROWS * HALF)])

    @pl.when(cid == 0)
    def _():
        run(rhA, oA)

    @pl.when(cid == 1)
    def _():
        run(rhB, oB)


def _make_agg_call():
    mesh = plsc.VectorSubcoreMesh(core_axis_name="c", subcore_axis_name="s")
    return pl.kernel(
        _agg_body,
        out_type=[jax.ShapeDtypeStruct((KP * HALF,), jnp.float32)] * 2,
        mesh=mesh,
        scratch_types=[
            pltpu.VMEM((2 * MCAP,), jnp.int32),
            pltpu.VMEM((NW * 16 + 16,), jnp.int32),
            pltpu.VMEM((2 * G,), jnp.int32),
            pltpu.VMEM((2 * G,), jnp.int32),
            pltpu.VMEM((G, HALF), jnp.float32),
            pltpu.VMEM((G, HALF), jnp.float32),
            pltpu.VMEM(((DROWS + 1) * HALF,), jnp.float32),
            pltpu.SemaphoreType.DMA,
            pltpu.SemaphoreType.DMA,
        ],
        compiler_params=pltpu.CompilerParams(needs_layout_passes=False),
    )


def _layer_a_body(hA, hB, aA, aB, w_ref, b_ref, eps_ref, z1_ref, st_ref):
    i = pl.program_id(0)
    h = jnp.concatenate([hA[...], hB[...]], axis=1)
    a = jnp.concatenate([aA[...], aB[...]], axis=1)
    zin = h * (1.0 + eps_ref[0]) + a
    z1 = jnp.dot(zin, w_ref[...], preferred_element_type=jnp.float32)
    z1 = z1 + b_ref[...]
    z1_ref[...] = z1

    @pl.when(i == 0)
    def _():
        st_ref[...] = jnp.zeros_like(st_ref)

    st_ref[0:1, :] = st_ref[0:1, :] + jnp.sum(z1, axis=0, keepdims=True)
    st_ref[1:2, :] = st_ref[1:2, :] + jnp.sum(z1 * z1, axis=0, keepdims=True)


def _layer_a_call(hA, hB, aA, aB, w1l, b1l, epsl):
    half_spec = pl.BlockSpec((RBLK, HALF), lambda i: (i, 0))
    return pl.pallas_call(
        _layer_a_body,
        grid=(K // RBLK,),
        in_specs=[half_spec] * 4 + [
            pl.BlockSpec((EMB, HID2), lambda i: (0, 0)),
            pl.BlockSpec((1, HID2), lambda i: (0, 0)),
            pl.BlockSpec(memory_space=pltpu.SMEM),
        ],
        out_specs=[
            pl.BlockSpec((RBLK, HID2), lambda i: (i, 0)),
            pl.BlockSpec((8, HID2), lambda i: (0, 0)),
        ],
        out_shape=[
            jax.ShapeDtypeStruct((K, HID2), jnp.float32),
            jax.ShapeDtypeStruct((8, HID2), jnp.float32),
        ],
    )(hA, hB, aA, aB, w1l, b1l, epsl)


def _layer_b_body(z1_ref, st_ref, g_ref, be_ref, w_ref, b_ref,
                  z2_ref, st2_ref):
    i = pl.program_id(0)
    kf = jnp.float32(K)
    mean = st_ref[0:1, :] / kf
    var = st_ref[1:2, :] / kf - mean * mean
    z1 = z1_ref[...]
    xb = g_ref[...] * (z1 - mean) / jnp.sqrt(var + 1e-5) + be_ref[...]
    y = jnp.maximum(xb, 0.0)
    z2 = jnp.dot(y, w_ref[...], preferred_element_type=jnp.float32)
    z2 = z2 + b_ref[...]
    z2_ref[...] = z2

    @pl.when(i == 0)
    def _():
        st2_ref[...] = jnp.zeros_like(st2_ref)

    st2_ref[0:1, :] = st2_ref[0:1, :] + jnp.sum(z2, axis=0, keepdims=True)
    st2_ref[1:2, :] = st2_ref[1:2, :] + jnp.sum(z2 * z2, axis=0, keepdims=True)


def _layer_b_call(z1, st, g1l, be1l, w2l, b2l):
    return pl.pallas_call(
        _layer_b_body,
        grid=(K // RBLK,),
        in_specs=[
            pl.BlockSpec((RBLK, HID2), lambda i: (i, 0)),
            pl.BlockSpec((8, HID2), lambda i: (0, 0)),
            pl.BlockSpec((1, HID2), lambda i: (0, 0)),
            pl.BlockSpec((1, HID2), lambda i: (0, 0)),
            pl.BlockSpec((HID2, EMB), lambda i: (0, 0)),
            pl.BlockSpec((1, EMB), lambda i: (0, 0)),
        ],
        out_specs=[
            pl.BlockSpec((RBLK, EMB), lambda i: (i, 0)),
            pl.BlockSpec((8, EMB), lambda i: (0, 0)),
        ],
        out_shape=[
            jax.ShapeDtypeStruct((K, EMB), jnp.float32),
            jax.ShapeDtypeStruct((8, EMB), jnp.float32),
        ],
    )(z1, st, g1l, be1l, w2l, b2l)


def _layer_c_body(z2_ref, st_ref, g_ref, be_ref, hA, hB):
    kf = jnp.float32(K)
    mean = st_ref[0:1, :] / kf
    var = st_ref[1:2, :] / kf - mean * mean
    xb = g_ref[...] * (z2_ref[...] - mean) / jnp.sqrt(var + 1e-5) + be_ref[...]
    h = jnp.maximum(xb, 0.0)
    hA[...] = h[:, :HALF]
    hB[...] = h[:, HALF:]


def _layer_c_call(z2, st2, gbnl, bbnl):
    half_spec = pl.BlockSpec((RBLK, HALF), lambda i: (i, 0))
    return pl.pallas_call(
        _layer_c_body,
        grid=(K // RBLK,),
        in_specs=[
            pl.BlockSpec((RBLK, EMB), lambda i: (i, 0)),
            pl.BlockSpec((8, EMB), lambda i: (0, 0)),
            pl.BlockSpec((1, EMB), lambda i: (0, 0)),
            pl.BlockSpec((1, EMB), lambda i: (0, 0)),
        ],
        out_specs=[half_spec] * 2,
        out_shape=[jax.ShapeDtypeStruct((K, HALF), jnp.float32)] * 2,
    )(z2, st2, gbnl, bbnl)


def _layer_pool_body(z2_ref, st_ref, g_ref, be_ref, pool_ref):
    i = pl.program_id(0)
    kf = jnp.float32(K)
    mean = st_ref[0:1, :] / kf
    var = st_ref[1:2, :] / kf - mean * mean
    h = g_ref[...] * (z2_ref[...] - mean) / jnp.sqrt(var + 1e-5) + be_ref[...]

    @pl.when(i == 0)
    def _():
        pool_ref[...] = jnp.zeros_like(pool_ref)

    pool_ref[0:1, :] = pool_ref[0:1, :] + jnp.sum(h, axis=0, keepdims=True)


def _layer_pool_call(z2, st2, gbnl, bbnl):
    return pl.pallas_call(
        _layer_pool_body,
        grid=(K // RBLK,),
        in_specs=[
            pl.BlockSpec((RBLK, EMB), lambda i: (i, 0)),
            pl.BlockSpec((8, EMB), lambda i: (0, 0)),
            pl.BlockSpec((1, EMB), lambda i: (0, 0)),
            pl.BlockSpec((1, EMB), lambda i: (0, 0)),
        ],
        out_specs=pl.BlockSpec((8, EMB), lambda i: (0, 0)),
        out_shape=jax.ShapeDtypeStruct((8, EMB), jnp.float32),
    )(z2, st2, gbnl, bbnl)


def _head_body(pool_ref, text_ref, wp1_ref, bp1_ref, wp2_ref, bp2_ref, o_ref):
    p = pool_ref[0:1, :] * (1.0 / jnp.float32(K))
    a1 = wp1_ref[0:EMB, :]
    a2 = wp1_ref[EMB:EMB + FEAT, :]
    r = jnp.dot(p, a1, preferred_element_type=jnp.float32)
    r = r + jnp.dot(text_ref[...], a2, preferred_element_type=jnp.float32)
    r = jnp.maximum(r + bp1_ref[...], 0.0)
    o = jnp.dot(r, wp2_ref[...], preferred_element_type=jnp.float32)
    o_ref[...] = o + bp2_ref[...]


def _head_call(pool, text_emb, wp1, bp1, wp2, bp2):
    return pl.pallas_call(
        _head_body,
        out_shape=jax.ShapeDtypeStruct((1, 2), jnp.float32),
    )(pool, text_emb, wp1, bp1, wp2, bp2)


def kernel(text_emb, demand_kg_emb, x, edge_index, W_fc, eps, W1, b1, g1,
           be1, W2, b2, gbn, bbn, Wp1, bp1, Wp2, bp2):
    score = _score_call(x, demand_kg_emb)
    scorep = jnp.concatenate(
        [score.reshape(N), jnp.full((NP - N,), -2.0, jnp.float32)]
    ).reshape(NP // 128, 128)
    sm2d, nm2d = _select_call(scorep)
    sm_col = sm2d.reshape(NP)[:N].reshape(N, 1)
    nm_flat = nm2d.reshape(NP)

    xsm = _prescale_call(x, sm_col)

    row = edge_index[0].astype(jnp.int32)
    col = edge_index[1].astype(jnp.int32)
    pad = jnp.full((E_PAD - E,), jnp.int32(N))
    rowp = jnp.concatenate([row, pad])
    colp = jnp.concatenate([col, pad])

    compact_fn = _make_compact_call()
    xs, elist, ecnt = compact_fn(xsm, nm_flat, rowp, colp)
    merge_fn = _make_merge_call()
    mlist, mcnt = merge_fn(elist, ecnt)

    hA, hB, rA, rB = _fc_call(xs, W_fc)

    zeros_sc = jnp.zeros((DROWS * HALF,), jnp.float32)
    agg_fn = _make_agg_call()

    for l in range(L):
        aAf, aBf = agg_fn(mlist, mcnt, zeros_sc, rA, rB)
        aA = aAf.reshape(KP, HALF)
        aB = aBf.reshape(KP, HALF)
        z1, st1 = _layer_a_call(hA, hB, aA, aB, W1[l],
                                b1[l].reshape(1, HID2), eps[l].reshape(1))
        z2, st2 = _layer_b_call(z1, st1, g1[l].reshape(1, HID2),
                                be1[l].reshape(1, HID2), W2[l],
                                b2[l].reshape(1, EMB))
        if l < L - 1:
            hA, hB = _layer_c_call(z2, st2, gbn[l].reshape(1, EMB),
                                   bbn[l].reshape(1, EMB))
            rA, rB = hA, hB
        else:
            pool = _layer_pool_call(z2, st2, gbn[l].reshape(1, EMB),
                                    bbn[l].reshape(1, EMB))

    return _head_call(pool, text_emb, Wp1, bp1.reshape(1, -1), Wp2,
                      bp2.reshape(1, 2))


# speedup vs baseline: 5.3311x; 1.5347x over previous
import functools
import math

import jax
import jax.numpy as jnp
from jax import lax
from jax.experimental import pallas as pl
from jax.experimental.pallas import tpu as pltpu
from jax.experimental.pallas import tpu_sc as plsc

N = 10000
E = 160000
FEAT = 256
EMB = 512
HALF = 256
HID2 = 2 * EMB
L = 3
K = int(math.ceil(0.5 * N))

NP = 10240
KP = 5120
RBLK = 1000
NSUB = 16
NCORE = 2
NW = NSUB * NCORE
E_PER_W = 5008
E_PAD = NW * E_PER_W
NODE_CHUNK = 80
N_CHUNKS = N // NODE_CHUNK
G = 64
BCAP = 256
DROWS = KP // NSUB
MCAP = 2048


def _score_body(x_ref, q_ref, o_ref):
    o_ref[...] = jnp.tanh(
        jnp.sum(x_ref[...] * q_ref[...], axis=1, keepdims=True))


def _score_call(x, q2d):
    return pl.pallas_call(
        _score_body,
        out_shape=jax.ShapeDtypeStruct((N, 1), jnp.float32),
    )(x, q2d)


def _select_body(s_ref, sm_ref, nm_ref):
    s = s_ref[...]
    bits = lax.bitcast_convert_type(s, jnp.int32)
    key = bits ^ ((bits >> 31) & jnp.int32(0x7FFFFFFF))
    kf = jnp.float32(K)

    def tbit(i, lo_u):
        b = 31 - i
        cand = lo_u | (jnp.int32(1) << b)
        t_s = cand ^ jnp.int32(-2147483648)
        cnt = jnp.sum((key >= t_s).astype(jnp.float32))
        return jnp.where(cnt >= kf, cand, lo_u)

    lo_u = lax.fori_loop(0, 32, tbit, jnp.int32(0))
    t_star = lo_u ^ jnp.int32(-2147483648)
    cnt_gt = jnp.sum((key > t_star).astype(jnp.float32))
    r = kf - cnt_gt

    rr = lax.broadcasted_iota(jnp.int32, s.shape, 0)
    cc = lax.broadcasted_iota(jnp.int32, s.shape, 1)
    idx = rr * 128 + cc
    eq = key == t_star

    def mbit(i, lo_m):
        b = 14 - i
        cand = lo_m | (jnp.int32(1) << b)
        ecnt = jnp.sum((eq & (idx < cand)).astype(jnp.float32))
        return jnp.where(ecnt <= r, cand, lo_m)

    m_star = lax.fori_loop(0, 15, mbit, jnp.int32(0))
    mask = (key > t_star) | (eq & (idx < m_star))
    maskf = mask.astype(jnp.float32)
    sm_ref[...] = maskf * s

    tri = (lax.broadcasted_iota(jnp.int32, (128, 128), 0)
           < lax.broadcasted_iota(jnp.int32, (128, 128), 1)).astype(jnp.float32)
    pre_in_row = jnp.dot(maskf, tri, preferred_element_type=jnp.float32)
    rows = s.shape[0]
    rs = jnp.sum(maskf, axis=1, keepdims=True)
    plow = (lax.broadcasted_iota(jnp.int32, (rows, rows), 1)
            < lax.broadcasted_iota(jnp.int32, (rows, rows), 0)
            ).astype(jnp.float32)
    row_off = jnp.dot(plow, rs, preferred_element_type=jnp.float32)
    c2d = row_off + pre_in_row
    nm_ref[...] = jnp.where(mask, c2d.astype(jnp.int32), jnp.int32(-1))


def _select_call(scorep):
    return pl.pallas_call(
        _select_body,
        out_shape=[
            jax.ShapeDtypeStruct((NP // 128, 128), jnp.float32),
            jax.ShapeDtypeStruct((NP // 128, 128), jnp.int32),
        ],
    )(scorep)


def _prescale_body(x_ref, sm_ref, o_ref):
    o_ref[...] = x_ref[...] * sm_ref[...]


def _prescale_call(x, sm_col):
    return pl.pallas_call(
        _prescale_body,
        grid=(5,),
        in_specs=[
            pl.BlockSpec((2000, FEAT), lambda i: (i, 0)),
            pl.BlockSpec((2000, 1), lambda i: (i, 0)),
        ],
        out_specs=pl.BlockSpec((2000, FEAT), lambda i: (i, 0)),
        out_shape=jax.ShapeDtypeStruct((N, FEAT), jnp.float32),
    )(x, sm_col)


def _compact_body(xsm_hbm, nm_hbm, rowp_hbm, colp_hbm,
                  xs_out, elist_out, ecnt_out,
                  nm_v, rowb, colb, bbuf, cntb, xrows, slotb, sem):
    cid = lax.axis_index("c")
    sid = lax.axis_index("s")
    w = sid * NCORE + cid
    pltpu.sync_copy(nm_hbm, nm_v)

    def do_chunk(ch):
        base = ch * NODE_CHUNK
        pltpu.sync_copy(xsm_hbm.at[pl.ds(base, NODE_CHUNK)], xrows)

        def lane(v, c2):
            nm16 = nm_v[pl.ds(base + v * 16, 16)]
            slotb[pl.ds(v * 16, 16)] = jnp.where(
                nm16 >= 0, nm16, jnp.int32(K) + w)
            return c2

        lax.fori_loop(0, NODE_CHUNK // 16, lane, 0)
        pltpu.sync_copy(xrows, xs_out.at[slotb])

    def chunk_loop(k, c2):
        ch = w + NW * k

        @pl.when(ch < N_CHUNKS)
        def _():
            do_chunk(ch)
        return c2

    lax.fori_loop(0, (N_CHUNKS + NW - 1) // NW, chunk_loop, 0)

    ebase = w * E_PER_W
    pltpu.sync_copy(rowp_hbm.at[pl.ds(ebase, E_PER_W)], rowb)
    pltpu.sync_copy(colp_hbm.at[pl.ds(ebase, E_PER_W)], colb)

    lane16 = lax.iota(jnp.int32, 16)

    def egroup(g, cntv):
        r16 = rowb[pl.ds(g * 16, 16)]
        c16 = colb[pl.ds(g * 16, 16)]
        mr = plsc.load_gather(nm_v, [r16])
        mc = plsc.load_gather(nm_v, [c16])
        keep = (mr >= 0) & (mc >= 0)
        packed = mr | (mc << 13)
        bucket = lax.div(mc, jnp.int32(DROWS))
        for b in range(NSUB):
            keepb = keep & (bucket == b)
            plsc.store_compressed(
                bbuf.at[b, pl.ds(cntv[b], 16)], packed, mask=keepb)
            onehot = (lane16 == b).astype(jnp.int32)
            cntv = cntv + onehot * jnp.sum(keepb.astype(jnp.int32))
        return cntv

    cntv = lax.fori_loop(0, E_PER_W // 16, egroup,
                         jnp.zeros((16,), jnp.int32))
    cntb[pl.ds(0, 16)] = cntv
    pltpu.sync_copy(bbuf, elist_out.at[w])
    pltpu.sync_copy(cntb, ecnt_out.at[pl.ds(w * 16, 16)])


def _make_compact_call():
    mesh = plsc.VectorSubcoreMesh(core_axis_name="c", subcore_axis_name="s")
    return pl.kernel(
        _compact_body,
        out_type=[
            jax.ShapeDtypeStruct((KP, FEAT), jnp.float32),
            jax.ShapeDtypeStruct((NW, NSUB, BCAP), jnp.int32),
            jax.ShapeDtypeStruct((NW * 16,), jnp.int32),
        ],
        mesh=mesh,
        scratch_types=[
            pltpu.VMEM((NP,), jnp.int32),
            pltpu.VMEM((E_PER_W,), jnp.int32),
            pltpu.VMEM((E_PER_W,), jnp.int32),
            pltpu.VMEM((NSUB, BCAP), jnp.int32),
            pltpu.VMEM((16,), jnp.int32),
            pltpu.VMEM((NODE_CHUNK, FEAT), jnp.float32),
            pltpu.VMEM((NODE_CHUNK,), jnp.int32),
            pltpu.SemaphoreType.DMA,
        ],
        compiler_params=pltpu.CompilerParams(needs_layout_passes=False),
    )


def _fc_body(x_ref, w_ref, hA, hB, rA, rB):
    z = jnp.dot(x_ref[...], w_ref[...], preferred_element_type=jnp.float32)
    zr = jnp.maximum(z, 0.0)
    hA[...] = z[:, :HALF]
    hB[...] = z[:, HALF:]
    rA[...] = zr[:, :HALF]
    rB[...] = zr[:, HALF:]


def _fc_call(xs, w_fc):
    half_spec = pl.BlockSpec((RBLK, HALF), lambda i: (i, 0))
    return pl.pallas_call(
        _fc_body,
        grid=(K // RBLK,),
        in_specs=[
            pl.BlockSpec((RBLK, FEAT), lambda i: (i, 0)),
            pl.BlockSpec((FEAT, EMB), lambda i: (0, 0)),
        ],
        out_specs=[half_spec] * 4,
        out_shape=[jax.ShapeDtypeStruct((K, HALF), jnp.float32)] * 4,
    )(xs, w_fc)


def _merge_body(elist_hbm, ecnt_hbm, mlist_out, mcnt_out,
                lbuf, cball, big, cntb, sem):
    cid = lax.axis_index("c")
    sid = lax.axis_index("s")
    pltpu.sync_copy(ecnt_hbm, cball.at[pl.ds(0, NW * 16)])
    lane16 = lax.iota(jnp.int32, 16)

    def wloop(wi, off):
        w = cid * NSUB + wi
        pltpu.sync_copy(elist_hbm.at[w, sid], lbuf)
        cnt = cball[pl.ds(w * 16 + sid, 16)][0]
        ng = lax.div(cnt + 15, jnp.int32(16))

        def cp(t, c2):
            big[pl.ds(off + t * 16, 16)] = lbuf[pl.ds(t * 16, 16)]
            return c2

        lax.fori_loop(0, ng, cp, 0)
        return off + cnt

    off = lax.fori_loop(0, NSUB, wloop, jnp.int32(0))
    padv = (lane16 * 0 + (sid * DROWS + DROWS)) << 13
    for t in range(G // 16):
        big[pl.ds(off + t * 16, 16)] = padv
    cntb[pl.ds(0, 16)] = lane16 * 0 + off
    seg = sid * 2 + cid
    pltpu.sync_copy(big, mlist_out.at[seg])
    pltpu.sync_copy(cntb, mcnt_out.at[pl.ds(seg * 16, 16)])


def _make_merge_call():
    mesh = plsc.VectorSubcoreMesh(core_axis_name="c", subcore_axis_name="s")
    return pl.kernel(
        _merge_body,
        out_type=[
            jax.ShapeDtypeStruct((NW, MCAP), jnp.int32),
            jax.ShapeDtypeStruct((NW * 16,), jnp.int32),
        ],
        mesh=mesh,
        scratch_types=[
            pltpu.VMEM((BCAP,), jnp.int32),
            pltpu.VMEM((NW * 16 + 16,), jnp.int32),
            pltpu.VMEM((MCAP,), jnp.int32),
            pltpu.VMEM((16,), jnp.int32),
            pltpu.SemaphoreType.DMA,
        ],
        compiler_params=pltpu.CompilerParams(needs_layout_passes=False),
    )


def _agg_body(mlist_hbm, mcnt_hbm, zeros_hbm, rhA, rhB, oA, oB,
              lbuf2, cball, srcb, offb, rows0, rows1, acc, sem0, sem1):
    cid = lax.axis_index("c")
    sid = lax.axis_index("s")

    pltpu.sync_copy(mcnt_hbm, cball.at[pl.ds(0, NW * 16)])
    pltpu.sync_copy(zeros_hbm, acc.at[pl.ds(0, DROWS * HALF)])
    pltpu.sync_copy(mlist_hbm.at[sid * 2], lbuf2.at[pl.ds(0, MCAP)])
    pltpu.sync_copy(mlist_hbm.at[sid * 2 + 1], lbuf2.at[pl.ds(MCAP, MCAP)])

    def unpack(pos, p):
        def sub_loop(sub, c):
            p16 = lbuf2[pl.ds(pos + sub * 16, 16)]
            srcb[pl.ds(p * G + sub * 16, 16)] = p16 & jnp.int32(0x1FFF)
            offb[pl.ds(p * G + sub * 16, 16)] = (
                (p16 >> 13) - sid * DROWS) * jnp.int32(HALF)
            return c

        lax.fori_loop(0, G // 16, sub_loop, 0)

    def start_gather(p, rows_ref, sem_ref):
        idx = srcb.at[pl.ds(p * G, G)]

        @pl.when(cid == 0)
        def _():
            pltpu.async_copy(rhA.at[idx], rows_ref, sem_ref)

        @pl.when(cid == 1)
        def _():
            pltpu.async_copy(rhB.at[idx], rows_ref, sem_ref)

    def wait_gather(p, rows_ref, sem_ref):
        pltpu.make_async_copy(
            rhA.at[srcb.at[pl.ds(p * G, G)]], rows_ref, sem_ref).wait()

    def accumulate(p, rows_ref):
        @functools.partial(plsc.parallel_loop, 0, G // 16)
        def sub_loop(sub):
            off16 = offb[pl.ds(p * G + sub * 16, 16)]
            for e in range(16):
                base = off16[e]
                for chunk in range(HALF // 16):
                    plsc.addupdate(
                        acc.at[pl.ds(base + chunk * 16, 16)],
                        rows_ref[sub * 16 + e, pl.ds(chunk * 16, 16)])

    def seg_loop(seg, c0):
        segbase = seg * MCAP
        cnt = cball[pl.ds(sid * 32 + seg * 16, 16)][0]
        ng = lax.div(cnt + (G - 1), jnp.int32(G))

        @pl.when(ng > 0)
        def _():
            unpack(segbase, 0)
            start_gather(0, rows0, sem0)

            def body(g, c2):
                @pl.when((g & 1) == 0)
                def _():
                    @pl.when(g + 1 < ng)
                    def _():
                        unpack(segbase + (g + 1) * G, 1)
                        start_gather(1, rows1, sem1)

                    wait_gather(0, rows0, sem0)
                    accumulate(0, rows0)

                @pl.when((g & 1) == 1)
                def _():
                    @pl.when(g + 1 < ng)
                    def _():
                        unpack(segbase + (g + 1) * G, 0)
                        start_gather(0, rows0, sem0)

                    wait_gather(1, rows1, sem1)
                    accumulate(1, rows1)

                return c2

            lax.fori_loop(0, ng, body, 0)

        return c0

    lax.fori_loop(0, 2, seg_loop, 0)
    out_slice = pl.ds(sid * DROWS * HALF, DROWS * HALF)

    @pl.when(cid == 0)
    def _():
        pltpu.sync_copy(acc.at[pl.ds(0, DROWS * HALF)], oA.at[out_slice])

    @pl.when(cid == 1)
    def _():
        pltpu.sync_copy(acc.at[pl.ds(0, DROWS * HALF)], oB.at[out_slice])


def _make_agg_call():
    mesh = plsc.VectorSubcoreMesh(core_axis_name="c", subcore_axis_name="s")
    return pl.kernel(
        _agg_body,
        out_type=[jax.ShapeDtypeStruct((KP * HALF,), jnp.float32)] * 2,
        mesh=mesh,
        scratch_types=[
            pltpu.VMEM((2 * MCAP,), jnp.int32),
            pltpu.VMEM((NW * 16 + 16,), jnp.int32),
            pltpu.VMEM((2 * G,), jnp.int32),
            pltpu.VMEM((2 * G,), jnp.int32),
            pltpu.VMEM((G, HALF), jnp.float32),
            pltpu.VMEM((G, HALF), jnp.float32),
            pltpu.VMEM(((DROWS + 1) * HALF,), jnp.float32),
            pltpu.SemaphoreType.DMA,
            pltpu.SemaphoreType.DMA,
        ],
        compiler_params=pltpu.CompilerParams(needs_layout_passes=False),
    )


def _layer_a_body(hA, hB, aA, aB, w_ref, b_ref, eps_ref, z1_ref, st_ref):
    i = pl.program_id(0)
    h = jnp.concatenate([hA[...], hB[...]], axis=1)
    a = jnp.concatenate([aA[...], aB[...]], axis=1)
    zin = h * (1.0 + eps_ref[0]) + a
    z1 = jnp.dot(zin, w_ref[...], preferred_element_type=jnp.float32)
    z1 = z1 + b_ref[...]
    z1_ref[...] = z1

    @pl.when(i == 0)
    def _():
        st_ref[...] = jnp.zeros_like(st_ref)

    st_ref[0:1, :] = st_ref[0:1, :] + jnp.sum(z1, axis=0, keepdims=True)
    st_ref[1:2, :] = st_ref[1:2, :] + jnp.sum(z1 * z1, axis=0, keepdims=True)


def _layer_a_call(hA, hB, aA, aB, w1l, b1l, epsl):
    half_spec = pl.BlockSpec((RBLK, HALF), lambda i: (i, 0))
    return pl.pallas_call(
        _layer_a_body,
        grid=(K // RBLK,),
        in_specs=[half_spec] * 4 + [
            pl.BlockSpec((EMB, HID2), lambda i: (0, 0)),
            pl.BlockSpec((1, HID2), lambda i: (0, 0)),
            pl.BlockSpec(memory_space=pltpu.SMEM),
        ],
        out_specs=[
            pl.BlockSpec((RBLK, HID2), lambda i: (i, 0)),
            pl.BlockSpec((8, HID2), lambda i: (0, 0)),
        ],
        out_shape=[
            jax.ShapeDtypeStruct((K, HID2), jnp.float32),
            jax.ShapeDtypeStruct((8, HID2), jnp.float32),
        ],
    )(hA, hB, aA, aB, w1l, b1l, epsl)


def _layer_b_body(z1_ref, st_ref, g_ref, be_ref, w_ref, b_ref,
                  z2_ref, st2_ref):
    i = pl.program_id(0)
    kf = jnp.float32(K)
    mean = st_ref[0:1, :] / kf
    var = st_ref[1:2, :] / kf - mean * mean
    z1 = z1_ref[...]
    xb = g_ref[...] * (z1 - mean) / jnp.sqrt(var + 1e-5) + be_ref[...]
    y = jnp.maximum(xb, 0.0)
    z2 = jnp.dot(y, w_ref[...], preferred_element_type=jnp.float32)
    z2 = z2 + b_ref[...]
    z2_ref[...] = z2

    @pl.when(i == 0)
    def _():
        st2_ref[...] = jnp.zeros_like(st2_ref)

    st2_ref[0:1, :] = st2_ref[0:1, :] + jnp.sum(z2, axis=0, keepdims=True)
    st2_ref[1:2, :] = st2_ref[1:2, :] + jnp.sum(z2 * z2, axis=0, keepdims=True)


def _layer_b_call(z1, st, g1l, be1l, w2l, b2l):
    return pl.pallas_call(
        _layer_b_body,
        grid=(K // RBLK,),
        in_specs=[
            pl.BlockSpec((RBLK, HID2), lambda i: (i, 0)),
            pl.BlockSpec((8, HID2), lambda i: (0, 0)),
            pl.BlockSpec((1, HID2), lambda i: (0, 0)),
            pl.BlockSpec((1, HID2), lambda i: (0, 0)),
            pl.BlockSpec((HID2, EMB), lambda i: (0, 0)),
            pl.BlockSpec((1, EMB), lambda i: (0, 0)),
        ],
        out_specs=[
            pl.BlockSpec((RBLK, EMB), lambda i: (i, 0)),
            pl.BlockSpec((8, EMB), lambda i: (0, 0)),
        ],
        out_shape=[
            jax.ShapeDtypeStruct((K, EMB), jnp.float32),
            jax.ShapeDtypeStruct((8, EMB), jnp.float32),
        ],
    )(z1, st, g1l, be1l, w2l, b2l)


def _layer_c_body(z2_ref, st_ref, g_ref, be_ref, hA, hB):
    kf = jnp.float32(K)
    mean = st_ref[0:1, :] / kf
    var = st_ref[1:2, :] / kf - mean * mean
    xb = g_ref[...] * (z2_ref[...] - mean) / jnp.sqrt(var + 1e-5) + be_ref[...]
    h = jnp.maximum(xb, 0.0)
    hA[...] = h[:, :HALF]
    hB[...] = h[:, HALF:]


def _layer_c_call(z2, st2, gbnl, bbnl):
    half_spec = pl.BlockSpec((RBLK, HALF), lambda i: (i, 0))
    return pl.pallas_call(
        _layer_c_body,
        grid=(K // RBLK,),
        in_specs=[
            pl.BlockSpec((RBLK, EMB), lambda i: (i, 0)),
            pl.BlockSpec((8, EMB), lambda i: (0, 0)),
            pl.BlockSpec((1, EMB), lambda i: (0, 0)),
            pl.BlockSpec((1, EMB), lambda i: (0, 0)),
        ],
        out_specs=[half_spec] * 2,
        out_shape=[jax.ShapeDtypeStruct((K, HALF), jnp.float32)] * 2,
    )(z2, st2, gbnl, bbnl)


def _layer_pool_body(z2_ref, st_ref, g_ref, be_ref, pool_ref):
    i = pl.program_id(0)
    kf = jnp.float32(K)
    mean = st_ref[0:1, :] / kf
    var = st_ref[1:2, :] / kf - mean * mean
    h = g_ref[...] * (z2_ref[...] - mean) / jnp.sqrt(var + 1e-5) + be_ref[...]

    @pl.when(i == 0)
    def _():
        pool_ref[...] = jnp.zeros_like(pool_ref)

    pool_ref[0:1, :] = pool_ref[0:1, :] + jnp.sum(h, axis=0, keepdims=True)


def _layer_pool_call(z2, st2, gbnl, bbnl):
    return pl.pallas_call(
        _layer_pool_body,
        grid=(K // RBLK,),
        in_specs=[
            pl.BlockSpec((RBLK, EMB), lambda i: (i, 0)),
            pl.BlockSpec((8, EMB), lambda i: (0, 0)),
            pl.BlockSpec((1, EMB), lambda i: (0, 0)),
            pl.BlockSpec((1, EMB), lambda i: (0, 0)),
        ],
        out_specs=pl.BlockSpec((8, EMB), lambda i: (0, 0)),
        out_shape=jax.ShapeDtypeStruct((8, EMB), jnp.float32),
    )(z2, st2, gbnl, bbnl)


def _head_body(pool_ref, text_ref, wp1_ref, bp1_ref, wp2_ref, bp2_ref, o_ref):
    p = pool_ref[0:1, :] * (1.0 / jnp.float32(K))
    a1 = wp1_ref[0:EMB, :]
    a2 = wp1_ref[EMB:EMB + FEAT, :]
    r = jnp.dot(p, a1, preferred_element_type=jnp.float32)
    r = r + jnp.dot(text_ref[...], a2, preferred_element_type=jnp.float32)
    r = jnp.maximum(r + bp1_ref[...], 0.0)
    o = jnp.dot(r, wp2_ref[...], preferred_element_type=jnp.float32)
    o_ref[...] = o + bp2_ref[...]


def _head_call(pool, text_emb, wp1, bp1, wp2, bp2):
    return pl.pallas_call(
        _head_body,
        out_shape=jax.ShapeDtypeStruct((1, 2), jnp.float32),
    )(pool, text_emb, wp1, bp1, wp2, bp2)


def kernel(text_emb, demand_kg_emb, x, edge_index, W_fc, eps, W1, b1, g1,
           be1, W2, b2, gbn, bbn, Wp1, bp1, Wp2, bp2):
    score = _score_call(x, demand_kg_emb)
    scorep = jnp.concatenate(
        [score.reshape(N), jnp.full((NP - N,), -2.0, jnp.float32)]
    ).reshape(NP // 128, 128)
    sm2d, nm2d = _select_call(scorep)
    sm_col = sm2d.reshape(NP)[:N].reshape(N, 1)
    nm_flat = nm2d.reshape(NP)

    xsm = _prescale_call(x, sm_col)

    row = edge_index[0].astype(jnp.int32)
    col = edge_index[1].astype(jnp.int32)
    pad = jnp.full((E_PAD - E,), jnp.int32(N))
    rowp = jnp.concatenate([row, pad])
    colp = jnp.concatenate([col, pad])

    compact_fn = _make_compact_call()
    xs, elist, ecnt = compact_fn(xsm, nm_flat, rowp, colp)
    merge_fn = _make_merge_call()
    mlist, mcnt = merge_fn(elist, ecnt)

    hA, hB, rA, rB = _fc_call(xs, W_fc)

    zeros_sc = jnp.zeros((DROWS * HALF,), jnp.float32)
    agg_fn = _make_agg_call()

    for l in range(L):
        aAf, aBf = agg_fn(mlist, mcnt, zeros_sc, rA, rB)
        aA = aAf.reshape(KP, HALF)
        aB = aBf.reshape(KP, HALF)
        z1, st1 = _layer_a_call(hA, hB, aA, aB, W1[l],
                                b1[l].reshape(1, HID2), eps[l].reshape(1))
        z2, st2 = _layer_b_call(z1, st1, g1[l].reshape(1, HID2),
                                be1[l].reshape(1, HID2), W2[l],
                                b2[l].reshape(1, EMB))
        if l < L - 1:
            hA, hB = _layer_c_call(z2, st2, gbn[l].reshape(1, EMB),
                                   bbn[l].reshape(1, EMB))
            rA, rB = hA, hB
        else:
            pool = _layer_pool_call(z2, st2, gbn[l].reshape(1, EMB),
                                    bbn[l].reshape(1, EMB))

    return _head_call(pool, text_emb, Wp1, bp1.reshape(1, -1), Wp2,
                      bp2.reshape(1, 2))


# speedup vs baseline: 5.8522x; 1.0977x over previous
import functools
import math

import jax
import jax.numpy as jnp
from jax import lax
from jax.experimental import pallas as pl
from jax.experimental.pallas import tpu as pltpu
from jax.experimental.pallas import tpu_sc as plsc

N = 10000
E = 160000
FEAT = 256
EMB = 512
HALF = 256
HID2 = 2 * EMB
L = 3
K = int(math.ceil(0.5 * N))

NP = 10240
KP = 5120
RBLK = 1000
NSUB = 16
NCORE = 2
NW = NSUB * NCORE
E_PER_W = 5008
E_PAD = NW * E_PER_W
NODE_CHUNK = 80
N_CHUNKS = N // NODE_CHUNK
G = 64
BCAP = 256
DROWS = KP // NSUB
MCAP = 2048


def _score_body(x_ref, q_ref, o_ref):
    o_ref[...] = jnp.tanh(
        jnp.sum(x_ref[...] * q_ref[...], axis=1, keepdims=True))


def _score_call(x, q2d):
    return pl.pallas_call(
        _score_body,
        out_shape=jax.ShapeDtypeStruct((N, 1), jnp.float32),
    )(x, q2d)


def _select_body(s_ref, sm_ref, nm_ref):
    s = s_ref[...]
    bits = lax.bitcast_convert_type(s, jnp.int32)
    key = bits ^ ((bits >> 31) & jnp.int32(0x7FFFFFFF))
    kf = jnp.float32(K)

    def tbit(i, lo_u):
        b = 31 - i
        cand = lo_u | (jnp.int32(1) << b)
        t_s = cand ^ jnp.int32(-2147483648)
        cnt = jnp.sum((key >= t_s).astype(jnp.float32))
        return jnp.where(cnt >= kf, cand, lo_u)

    lo_u = lax.fori_loop(0, 32, tbit, jnp.int32(0))
    t_star = lo_u ^ jnp.int32(-2147483648)
    cnt_gt = jnp.sum((key > t_star).astype(jnp.float32))
    r = kf - cnt_gt

    rr = lax.broadcasted_iota(jnp.int32, s.shape, 0)
    cc = lax.broadcasted_iota(jnp.int32, s.shape, 1)
    idx = rr * 128 + cc
    eq = key == t_star

    def mbit(i, lo_m):
        b = 14 - i
        cand = lo_m | (jnp.int32(1) << b)
        ecnt = jnp.sum((eq & (idx < cand)).astype(jnp.float32))
        return jnp.where(ecnt <= r, cand, lo_m)

    m_star = lax.fori_loop(0, 15, mbit, jnp.int32(0))
    mask = (key > t_star) | (eq & (idx < m_star))
    maskf = mask.astype(jnp.float32)
    sm_ref[...] = maskf * s

    tri = (lax.broadcasted_iota(jnp.int32, (128, 128), 0)
           < lax.broadcasted_iota(jnp.int32, (128, 128), 1)).astype(jnp.float32)
    pre_in_row = jnp.dot(maskf, tri, preferred_element_type=jnp.float32)
    rows = s.shape[0]
    rs = jnp.sum(maskf, axis=1, keepdims=True)
    plow = (lax.broadcasted_iota(jnp.int32, (rows, rows), 1)
            < lax.broadcasted_iota(jnp.int32, (rows, rows), 0)
            ).astype(jnp.float32)
    row_off = jnp.dot(plow, rs, preferred_element_type=jnp.float32)
    c2d = row_off + pre_in_row
    nm_ref[...] = jnp.where(mask, c2d.astype(jnp.int32), jnp.int32(-1))


def _select_call(scorep):
    return pl.pallas_call(
        _select_body,
        out_shape=[
            jax.ShapeDtypeStruct((NP // 128, 128), jnp.float32),
            jax.ShapeDtypeStruct((NP // 128, 128), jnp.int32),
        ],
    )(scorep)


def _prescale_body(x_ref, sm_ref, o_ref):
    o_ref[...] = x_ref[...] * sm_ref[...]


def _prescale_call(x, sm_col):
    return pl.pallas_call(
        _prescale_body,
        grid=(5,),
        in_specs=[
            pl.BlockSpec((2000, FEAT), lambda i: (i, 0)),
            pl.BlockSpec((2000, 1), lambda i: (i, 0)),
        ],
        out_specs=pl.BlockSpec((2000, FEAT), lambda i: (i, 0)),
        out_shape=jax.ShapeDtypeStruct((N, FEAT), jnp.float32),
    )(x, sm_col)


def _compact_body(xsm_hbm, nm_hbm, rowp_hbm, colp_hbm,
                  xs_out, elist_out, ecnt_out,
                  nm_v, rowb, colb, bbuf, cntb, xrows, slotb, sem):
    cid = lax.axis_index("c")
    sid = lax.axis_index("s")
    w = sid * NCORE + cid
    pltpu.sync_copy(nm_hbm, nm_v)

    def do_chunk(ch):
        base = ch * NODE_CHUNK
        pltpu.sync_copy(xsm_hbm.at[pl.ds(base, NODE_CHUNK)], xrows)

        def lane(v, c2):
            nm16 = nm_v[pl.ds(base + v * 16, 16)]
            slotb[pl.ds(v * 16, 16)] = jnp.where(
                nm16 >= 0, nm16, jnp.int32(K) + w)
            return c2

        lax.fori_loop(0, NODE_CHUNK // 16, lane, 0)
        pltpu.sync_copy(xrows, xs_out.at[slotb])

    def chunk_loop(k, c2):
        ch = w + NW * k

        @pl.when(ch < N_CHUNKS)
        def _():
            do_chunk(ch)
        return c2

    lax.fori_loop(0, (N_CHUNKS + NW - 1) // NW, chunk_loop, 0)

    ebase = w * E_PER_W
    pltpu.sync_copy(rowp_hbm.at[pl.ds(ebase, E_PER_W)], rowb)
    pltpu.sync_copy(colp_hbm.at[pl.ds(ebase, E_PER_W)], colb)

    lane16 = lax.iota(jnp.int32, 16)

    def egroup(g, cntv):
        r16 = rowb[pl.ds(g * 16, 16)]
        c16 = colb[pl.ds(g * 16, 16)]
        mr = plsc.load_gather(nm_v, [r16])
        mc = plsc.load_gather(nm_v, [c16])
        keep = (mr >= 0) & (mc >= 0)
        packed = mr | (mc << 13)
        bucket = lax.div(mc, jnp.int32(DROWS))
        for b in range(NSUB):
            keepb = keep & (bucket == b)
            plsc.store_compressed(
                bbuf.at[b, pl.ds(cntv[b], 16)], packed, mask=keepb)
            onehot = (lane16 == b).astype(jnp.int32)
            cntv = cntv + onehot * jnp.sum(keepb.astype(jnp.int32))
        return cntv

    cntv = lax.fori_loop(0, E_PER_W // 16, egroup,
                         jnp.zeros((16,), jnp.int32))
    cntb[pl.ds(0, 16)] = cntv
    pltpu.sync_copy(bbuf, elist_out.at[w])
    pltpu.sync_copy(cntb, ecnt_out.at[pl.ds(w * 16, 16)])


def _make_compact_call():
    mesh = plsc.VectorSubcoreMesh(core_axis_name="c", subcore_axis_name="s")
    return pl.kernel(
        _compact_body,
        out_type=[
            jax.ShapeDtypeStruct((KP, FEAT), jnp.float32),
            jax.ShapeDtypeStruct((NW, NSUB, BCAP), jnp.int32),
            jax.ShapeDtypeStruct((NW * 16,), jnp.int32),
        ],
        mesh=mesh,
        scratch_types=[
            pltpu.VMEM((NP,), jnp.int32),
            pltpu.VMEM((E_PER_W,), jnp.int32),
            pltpu.VMEM((E_PER_W,), jnp.int32),
            pltpu.VMEM((NSUB, BCAP), jnp.int32),
            pltpu.VMEM((16,), jnp.int32),
            pltpu.VMEM((NODE_CHUNK, FEAT), jnp.float32),
            pltpu.VMEM((NODE_CHUNK,), jnp.int32),
            pltpu.SemaphoreType.DMA,
        ],
        compiler_params=pltpu.CompilerParams(needs_layout_passes=False),
    )


def _fc_body(x_ref, w_ref, hA, hB, rA, rB):
    z = jnp.dot(x_ref[...], w_ref[...], preferred_element_type=jnp.float32)
    zr = jnp.maximum(z, 0.0)
    hA[...] = z[:, :HALF]
    hB[...] = z[:, HALF:]
    rA[...] = zr[:, :HALF]
    rB[...] = zr[:, HALF:]


def _fc_call(xs, w_fc):
    half_spec = pl.BlockSpec((RBLK, HALF), lambda i: (i, 0))
    return pl.pallas_call(
        _fc_body,
        grid=(K // RBLK,),
        in_specs=[
            pl.BlockSpec((RBLK, FEAT), lambda i: (i, 0)),
            pl.BlockSpec((FEAT, EMB), lambda i: (0, 0)),
        ],
        out_specs=[half_spec] * 4,
        out_shape=[jax.ShapeDtypeStruct((K, HALF), jnp.float32)] * 4,
    )(xs, w_fc)


def _merge_body(elist_hbm, ecnt_hbm, mlist_out, mcnt_out,
                lbuf, cball, big, cntb, sem):
    cid = lax.axis_index("c")
    sid = lax.axis_index("s")
    pltpu.sync_copy(ecnt_hbm, cball.at[pl.ds(0, NW * 16)])
    lane16 = lax.iota(jnp.int32, 16)

    def wloop(wi, off):
        w = cid * NSUB + wi
        pltpu.sync_copy(elist_hbm.at[w, sid], lbuf)
        cnt = cball[pl.ds(w * 16 + sid, 16)][0]
        ng = lax.div(cnt + 15, jnp.int32(16))

        def cp(t, c2):
            big[pl.ds(off + t * 16, 16)] = lbuf[pl.ds(t * 16, 16)]
            return c2

        lax.fori_loop(0, ng, cp, 0)
        return off + cnt

    off = lax.fori_loop(0, NSUB, wloop, jnp.int32(0))
    padv = (lane16 * 0 + (sid * DROWS + DROWS)) << 13
    for t in range(G // 16):
        big[pl.ds(off + t * 16, 16)] = padv
    cntb[pl.ds(0, 16)] = lane16 * 0 + off
    seg = sid * 2 + cid
    pltpu.sync_copy(big, mlist_out.at[seg])
    pltpu.sync_copy(cntb, mcnt_out.at[pl.ds(seg * 16, 16)])


def _make_merge_call():
    mesh = plsc.VectorSubcoreMesh(core_axis_name="c", subcore_axis_name="s")
    return pl.kernel(
        _merge_body,
        out_type=[
            jax.ShapeDtypeStruct((NW, MCAP), jnp.int32),
            jax.ShapeDtypeStruct((NW * 16,), jnp.int32),
        ],
        mesh=mesh,
        scratch_types=[
            pltpu.VMEM((BCAP,), jnp.int32),
            pltpu.VMEM((NW * 16 + 16,), jnp.int32),
            pltpu.VMEM((MCAP,), jnp.int32),
            pltpu.VMEM((16,), jnp.int32),
            pltpu.SemaphoreType.DMA,
        ],
        compiler_params=pltpu.CompilerParams(needs_layout_passes=False),
    )


def _agg_body(mlist_hbm, mcnt_hbm, zeros_hbm, rhA, rhB, oA, oB,
              lbuf2, cball, srcb, offb, rows0, rows1, acc, sem0, sem1):
    cid = lax.axis_index("c")
    sid = lax.axis_index("s")

    pltpu.sync_copy(mcnt_hbm, cball.at[pl.ds(0, NW * 16)])
    pltpu.sync_copy(zeros_hbm, acc.at[pl.ds(0, DROWS * HALF)])
    pltpu.sync_copy(mlist_hbm.at[sid * 2], lbuf2.at[pl.ds(0, MCAP)])
    pltpu.sync_copy(mlist_hbm.at[sid * 2 + 1], lbuf2.at[pl.ds(MCAP, MCAP)])

    def unpack(pos, p):
        def sub_loop(sub, c):
            p16 = lbuf2[pl.ds(pos + sub * 16, 16)]
            srcb[pl.ds(p * G + sub * 16, 16)] = p16 & jnp.int32(0x1FFF)
            offb[pl.ds(p * G + sub * 16, 16)] = (
                (p16 >> 13) - sid * DROWS) * jnp.int32(HALF)
            return c

        lax.fori_loop(0, G // 16, sub_loop, 0)

    def start_gather(p, rows_ref, sem_ref):
        idx = srcb.at[pl.ds(p * G, G)]

        @pl.when(cid == 0)
        def _():
            pltpu.async_copy(rhA.at[idx], rows_ref, sem_ref)

        @pl.when(cid == 1)
        def _():
            pltpu.async_copy(rhB.at[idx], rows_ref, sem_ref)

    def wait_gather(p, rows_ref, sem_ref):
        pltpu.make_async_copy(
            rhA.at[srcb.at[pl.ds(p * G, G)]], rows_ref, sem_ref).wait()

    def accumulate(p, rows_ref):
        @functools.partial(plsc.parallel_loop, 0, G // 16)
        def sub_loop(sub):
            off16 = offb[pl.ds(p * G + sub * 16, 16)]
            for e in range(16):
                base = off16[e]
                for chunk in range(HALF // 16):
                    plsc.addupdate(
                        acc.at[pl.ds(base + chunk * 16, 16)],
                        rows_ref[sub * 16 + e, pl.ds(chunk * 16, 16)])

    def seg_loop(seg, c0):
        segbase = seg * MCAP
        cnt = cball[pl.ds(sid * 32 + seg * 16, 16)][0]
        ng = lax.div(cnt + (G - 1), jnp.int32(G))

        @pl.when(ng > 0)
        def _():
            unpack(segbase, 0)
            start_gather(0, rows0, sem0)

            def body(g, c2):
                @pl.when((g & 1) == 0)
                def _():
                    @pl.when(g + 1 < ng)
                    def _():
                        unpack(segbase + (g + 1) * G, 1)
                        start_gather(1, rows1, sem1)

                    wait_gather(0, rows0, sem0)
                    accumulate(0, rows0)

                @pl.when((g & 1) == 1)
                def _():
                    @pl.when(g + 1 < ng)
                    def _():
                        unpack(segbase + (g + 1) * G, 0)
                        start_gather(0, rows0, sem0)

                    wait_gather(1, rows1, sem1)
                    accumulate(1, rows1)

                return c2

            lax.fori_loop(0, ng, body, 0)

        return c0

    lax.fori_loop(0, 2, seg_loop, 0)
    out_slice = pl.ds(sid * DROWS * HALF, DROWS * HALF)

    @pl.when(cid == 0)
    def _():
        pltpu.sync_copy(acc.at[pl.ds(0, DROWS * HALF)], oA.at[out_slice])

    @pl.when(cid == 1)
    def _():
        pltpu.sync_copy(acc.at[pl.ds(0, DROWS * HALF)], oB.at[out_slice])


def _make_agg_call():
    mesh = plsc.VectorSubcoreMesh(core_axis_name="c", subcore_axis_name="s")
    return pl.kernel(
        _agg_body,
        out_type=[jax.ShapeDtypeStruct((KP * HALF,), jnp.float32)] * 2,
        mesh=mesh,
        scratch_types=[
            pltpu.VMEM((2 * MCAP,), jnp.int32),
            pltpu.VMEM((NW * 16 + 16,), jnp.int32),
            pltpu.VMEM((2 * G,), jnp.int32),
            pltpu.VMEM((2 * G,), jnp.int32),
            pltpu.VMEM((G, HALF), jnp.float32),
            pltpu.VMEM((G, HALF), jnp.float32),
            pltpu.VMEM(((DROWS + 1) * HALF,), jnp.float32),
            pltpu.SemaphoreType.DMA,
            pltpu.SemaphoreType.DMA,
        ],
        compiler_params=pltpu.CompilerParams(needs_layout_passes=False),
    )


def _layer_fused_body(hA, hB, aA, aB, w1_ref, b1_ref, g1_ref, be1_ref,
                      w2_ref, b2_ref, gbn_ref, bbn_ref, eps_ref,
                      outA, outB, z1_s, z2_s, st1_s, st2_s, *, last):
    p = pl.program_id(0)
    i = pl.program_id(1)
    kf = jnp.float32(K)

    @pl.when((p == 0) & (i == 0))
    def _():
        st1_s[...] = jnp.zeros_like(st1_s)
        st2_s[...] = jnp.zeros_like(st2_s)

    @pl.when(p == 0)
    def _():
        h = jnp.concatenate([hA[...], hB[...]], axis=1)
        a = jnp.concatenate([aA[...], aB[...]], axis=1)
        zin = h * (1.0 + eps_ref[0]) + a
        z1 = jnp.dot(zin, w1_ref[...], preferred_element_type=jnp.float32)
        z1 = z1 + b1_ref[...]
        z1_s[pl.ds(i * RBLK, RBLK), :] = z1
        st1_s[0:1, :] = st1_s[0:1, :] + jnp.sum(z1, axis=0, keepdims=True)
        st1_s[1:2, :] = st1_s[1:2, :] + jnp.sum(z1 * z1, axis=0,
                                                keepdims=True)

    @pl.when(p == 1)
    def _():
        mean = st1_s[0:1, :] / kf
        var = st1_s[1:2, :] / kf - mean * mean
        z1 = z1_s[pl.ds(i * RBLK, RBLK), :]
        xb = g1_ref[...] * (z1 - mean) / jnp.sqrt(var + 1e-5) + be1_ref[...]
        y = jnp.maximum(xb, 0.0)
        z2 = jnp.dot(y, w2_ref[...], preferred_element_type=jnp.float32)
        z2 = z2 + b2_ref[...]
        z2_s[pl.ds(i * RBLK, RBLK), :] = z2
        st2_s[0:1, :] = st2_s[0:1, :] + jnp.sum(z2, axis=0, keepdims=True)
        st2_s[1:2, :] = st2_s[1:2, :] + jnp.sum(z2 * z2, axis=0,
                                                keepdims=True)

    @pl.when(p == 2)
    def _():
        mean = st2_s[0:1, :] / kf
        var = st2_s[1:2, :] / kf - mean * mean
        z2 = z2_s[pl.ds(i * RBLK, RBLK), :]
        h = gbn_ref[...] * (z2 - mean) / jnp.sqrt(var + 1e-5) + bbn_ref[...]
        if last:
            @pl.when(i == 0)
            def _():
                outA[...] = jnp.zeros_like(outA)

            outA[0:1, :] = outA[0:1, :] + jnp.sum(h, axis=0, keepdims=True)
        else:
            h = jnp.maximum(h, 0.0)
            outA[...] = h[:, :HALF]
            outB[...] = h[:, HALF:]


def _layer_fused_call(hA, hB, aA, aB, w1l, b1l, g1l, be1l, w2l, b2l,
                      gbnl, bbnl, epsl, last):
    nblk = K // RBLK

    def blk0(p, i):
        return (jnp.where(p == 0, i, 0), 0)

    def blk2(p, i):
        return (jnp.where(p == 2, i, 0), 0)

    const = pl.BlockSpec((1, HID2), lambda p, i: (0, 0))
    const2 = pl.BlockSpec((1, EMB), lambda p, i: (0, 0))
    if last:
        out_specs = [pl.BlockSpec((8, EMB), lambda p, i: (0, 0)),
                     pl.BlockSpec((8, EMB), lambda p, i: (0, 0))]
        out_shape = [jax.ShapeDtypeStruct((8, EMB), jnp.float32)] * 2
    else:
        out_specs = [pl.BlockSpec((RBLK, HALF), blk2)] * 2
        out_shape = [jax.ShapeDtypeStruct((K, HALF), jnp.float32)] * 2
    return pl.pallas_call(
        functools.partial(_layer_fused_body, last=last),
        grid=(3, nblk),
        in_specs=[
            pl.BlockSpec((RBLK, HALF), blk0),
            pl.BlockSpec((RBLK, HALF), blk0),
            pl.BlockSpec((RBLK, HALF), blk0),
            pl.BlockSpec((RBLK, HALF), blk0),
            pl.BlockSpec((EMB, HID2), lambda p, i: (0, 0)),
            const,
            const,
            const,
            pl.BlockSpec((HID2, EMB), lambda p, i: (0, 0)),
            const2,
            const2,
            const2,
            pl.BlockSpec(memory_space=pltpu.SMEM),
        ],
        out_specs=out_specs,
        out_shape=out_shape,
        scratch_shapes=[
            pltpu.VMEM((K, HID2), jnp.float32),
            pltpu.VMEM((K, EMB), jnp.float32),
            pltpu.VMEM((8, HID2), jnp.float32),
            pltpu.VMEM((8, EMB), jnp.float32),
        ],
    )(hA, hB, aA, aB, w1l, b1l, g1l, be1l, w2l, b2l, gbnl, bbnl, epsl)


def _layer_a_body(hA, hB, aA, aB, w_ref, b_ref, eps_ref, z1_ref, st_ref):
    i = pl.program_id(0)
    h = jnp.concatenate([hA[...], hB[...]], axis=1)
    a = jnp.concatenate([aA[...], aB[...]], axis=1)
    zin = h * (1.0 + eps_ref[0]) + a
    z1 = jnp.dot(zin, w_ref[...], preferred_element_type=jnp.float32)
    z1 = z1 + b_ref[...]
    z1_ref[...] = z1

    @pl.when(i == 0)
    def _():
        st_ref[...] = jnp.zeros_like(st_ref)

    st_ref[0:1, :] = st_ref[0:1, :] + jnp.sum(z1, axis=0, keepdims=True)
    st_ref[1:2, :] = st_ref[1:2, :] + jnp.sum(z1 * z1, axis=0, keepdims=True)


def _layer_a_call(hA, hB, aA, aB, w1l, b1l, epsl):
    half_spec = pl.BlockSpec((RBLK, HALF), lambda i: (i, 0))
    return pl.pallas_call(
        _layer_a_body,
        grid=(K // RBLK,),
        in_specs=[half_spec] * 4 + [
            pl.BlockSpec((EMB, HID2), lambda i: (0, 0)),
            pl.BlockSpec((1, HID2), lambda i: (0, 0)),
            pl.BlockSpec(memory_space=pltpu.SMEM),
        ],
        out_specs=[
            pl.BlockSpec((RBLK, HID2), lambda i: (i, 0)),
            pl.BlockSpec((8, HID2), lambda i: (0, 0)),
        ],
        out_shape=[
            jax.ShapeDtypeStruct((K, HID2), jnp.float32),
            jax.ShapeDtypeStruct((8, HID2), jnp.float32),
        ],
    )(hA, hB, aA, aB, w1l, b1l, epsl)


def _layer_b_body(z1_ref, st_ref, g_ref, be_ref, w_ref, b_ref,
                  z2_ref, st2_ref):
    i = pl.program_id(0)
    kf = jnp.float32(K)
    mean = st_ref[0:1, :] / kf
    var = st_ref[1:2, :] / kf - mean * mean
    z1 = z1_ref[...]
    xb = g_ref[...] * (z1 - mean) / jnp.sqrt(var + 1e-5) + be_ref[...]
    y = jnp.maximum(xb, 0.0)
    z2 = jnp.dot(y, w_ref[...], preferred_element_type=jnp.float32)
    z2 = z2 + b_ref[...]
    z2_ref[...] = z2

    @pl.when(i == 0)
    def _():
        st2_ref[...] = jnp.zeros_like(st2_ref)

    st2_ref[0:1, :] = st2_ref[0:1, :] + jnp.sum(z2, axis=0, keepdims=True)
    st2_ref[1:2, :] = st2_ref[1:2, :] + jnp.sum(z2 * z2, axis=0, keepdims=True)


def _layer_b_call(z1, st, g1l, be1l, w2l, b2l):
    return pl.pallas_call(
        _layer_b_body,
        grid=(K // RBLK,),
        in_specs=[
            pl.BlockSpec((RBLK, HID2), lambda i: (i, 0)),
            pl.BlockSpec((8, HID2), lambda i: (0, 0)),
            pl.BlockSpec((1, HID2), lambda i: (0, 0)),
            pl.BlockSpec((1, HID2), lambda i: (0, 0)),
            pl.BlockSpec((HID2, EMB), lambda i: (0, 0)),
            pl.BlockSpec((1, EMB), lambda i: (0, 0)),
        ],
        out_specs=[
            pl.BlockSpec((RBLK, EMB), lambda i: (i, 0)),
            pl.BlockSpec((8, EMB), lambda i: (0, 0)),
        ],
        out_shape=[
            jax.ShapeDtypeStruct((K, EMB), jnp.float32),
            jax.ShapeDtypeStruct((8, EMB), jnp.float32),
        ],
    )(z1, st, g1l, be1l, w2l, b2l)


def _layer_c_body(z2_ref, st_ref, g_ref, be_ref, hA, hB):
    kf = jnp.float32(K)
    mean = st_ref[0:1, :] / kf
    var = st_ref[1:2, :] / kf - mean * mean
    xb = g_ref[...] * (z2_ref[...] - mean) / jnp.sqrt(var + 1e-5) + be_ref[...]
    h = jnp.maximum(xb, 0.0)
    hA[...] = h[:, :HALF]
    hB[...] = h[:, HALF:]


def _layer_c_call(z2, st2, gbnl, bbnl):
    half_spec = pl.BlockSpec((RBLK, HALF), lambda i: (i, 0))
    return pl.pallas_call(
        _layer_c_body,
        grid=(K // RBLK,),
        in_specs=[
            pl.BlockSpec((RBLK, EMB), lambda i: (i, 0)),
            pl.BlockSpec((8, EMB), lambda i: (0, 0)),
            pl.BlockSpec((1, EMB), lambda i: (0, 0)),
            pl.BlockSpec((1, EMB), lambda i: (0, 0)),
        ],
        out_specs=[half_spec] * 2,
        out_shape=[jax.ShapeDtypeStruct((K, HALF), jnp.float32)] * 2,
    )(z2, st2, gbnl, bbnl)


def _layer_pool_body(z2_ref, st_ref, g_ref, be_ref, pool_ref):
    i = pl.program_id(0)
    kf = jnp.float32(K)
    mean = st_ref[0:1, :] / kf
    var = st_ref[1:2, :] / kf - mean * mean
    h = g_ref[...] * (z2_ref[...] - mean) / jnp.sqrt(var + 1e-5) + be_ref[...]

    @pl.when(i == 0)
    def _():
        pool_ref[...] = jnp.zeros_like(pool_ref)

    pool_ref[0:1, :] = pool_ref[0:1, :] + jnp.sum(h, axis=0, keepdims=True)


def _layer_pool_call(z2, st2, gbnl, bbnl):
    return pl.pallas_call(
        _layer_pool_body,
        grid=(K // RBLK,),
        in_specs=[
            pl.BlockSpec((RBLK, EMB), lambda i: (i, 0)),
            pl.BlockSpec((8, EMB), lambda i: (0, 0)),
            pl.BlockSpec((1, EMB), lambda i: (0, 0)),
            pl.BlockSpec((1, EMB), lambda i: (0, 0)),
        ],
        out_specs=pl.BlockSpec((8, EMB), lambda i: (0, 0)),
        out_shape=jax.ShapeDtypeStruct((8, EMB), jnp.float32),
    )(z2, st2, gbnl, bbnl)


def _head_body(pool_ref, text_ref, wp1_ref, bp1_ref, wp2_ref, bp2_ref, o_ref):
    p = pool_ref[0:1, :] * (1.0 / jnp.float32(K))
    a1 = wp1_ref[0:EMB, :]
    a2 = wp1_ref[EMB:EMB + FEAT, :]
    r = jnp.dot(p, a1, preferred_element_type=jnp.float32)
    r = r + jnp.dot(text_ref[...], a2, preferred_element_type=jnp.float32)
    r = jnp.maximum(r + bp1_ref[...], 0.0)
    o = jnp.dot(r, wp2_ref[...], preferred_element_type=jnp.float32)
    o_ref[...] = o + bp2_ref[...]


def _head_call(pool, text_emb, wp1, bp1, wp2, bp2):
    return pl.pallas_call(
        _head_body,
        out_shape=jax.ShapeDtypeStruct((1, 2), jnp.float32),
    )(pool, text_emb, wp1, bp1, wp2, bp2)


def kernel(text_emb, demand_kg_emb, x, edge_index, W_fc, eps, W1, b1, g1,
           be1, W2, b2, gbn, bbn, Wp1, bp1, Wp2, bp2):
    score = _score_call(x, demand_kg_emb)
    scorep = jnp.concatenate(
        [score.reshape(N), jnp.full((NP - N,), -2.0, jnp.float32)]
    ).reshape(NP // 128, 128)
    sm2d, nm2d = _select_call(scorep)
    sm_col = sm2d.reshape(NP)[:N].reshape(N, 1)
    nm_flat = nm2d.reshape(NP)

    xsm = _prescale_call(x, sm_col)

    row = edge_index[0].astype(jnp.int32)
    col = edge_index[1].astype(jnp.int32)
    pad = jnp.full((E_PAD - E,), jnp.int32(N))
    rowp = jnp.concatenate([row, pad])
    colp = jnp.concatenate([col, pad])

    compact_fn = _make_compact_call()
    xs, elist, ecnt = compact_fn(xsm, nm_flat, rowp, colp)
    merge_fn = _make_merge_call()
    mlist, mcnt = merge_fn(elist, ecnt)

    hA, hB, rA, rB = _fc_call(xs, W_fc)

    zeros_sc = jnp.zeros((DROWS * HALF,), jnp.float32)
    agg_fn = _make_agg_call()

    for l in range(L):
        aAf, aBf = agg_fn(mlist, mcnt, zeros_sc, rA, rB)
        aA = aAf.reshape(KP, HALF)
        aB = aBf.reshape(KP, HALF)
        outs = _layer_fused_call(
            hA, hB, aA, aB, W1[l], b1[l].reshape(1, HID2),
            g1[l].reshape(1, HID2), be1[l].reshape(1, HID2), W2[l],
            b2[l].reshape(1, EMB), gbn[l].reshape(1, EMB),
            bbn[l].reshape(1, EMB), eps[l].reshape(1), last=(l == L - 1))
        if l < L - 1:
            hA, hB = outs
            rA, rB = hA, hB
        else:
            pool = outs[0]

    return _head_call(pool, text_emb, Wp1, bp1.reshape(1, -1), Wp2,
                      bp2.reshape(1, 2))
